# baseline (speedup 1.0000x reference)
import functools

import jax
import jax.numpy as jnp
from jax import lax
from jax.experimental import pallas as pl
from jax.experimental.pallas import tpu as pltpu
from jax.experimental.pallas import tpu_sc as plsc

N = 10000
E = 160000
EL = E + N
DIN = 128
HID = 512
H = 8
C = 64
G = 16
NC = 5
NB = 25
BR = N // NB

_HIGH = lax.Precision.HIGHEST


def _proj_body(x_ref, wp_ref, bp_ref, o_ref):
    o_ref[...] = jax.nn.relu(
        jnp.dot(x_ref[...], wp_ref[...], precision=_HIGH,
                preferred_element_type=jnp.float32) + bp_ref[...])


def _proj(x, Wp, bp):
    return pl.pallas_call(
        _proj_body,
        grid=(NB,),
        in_specs=[pl.BlockSpec((BR, DIN), lambda i: (i, 0)),
                  pl.BlockSpec((DIN, HID), lambda i: (0, 0)),
                  pl.BlockSpec((1, HID), lambda i: (0, 0))],
        out_specs=pl.BlockSpec((BR, HID), lambda i: (i, 0)),
        out_shape=jax.ShapeDtypeStruct((N, HID), jnp.float32),
    )(x, Wp, bp.reshape(1, HID))


def _layermm_body(h_ref, w_ref, a_ref, xl_ref, al_ref):
    xl = jnp.dot(h_ref[...], w_ref[...], precision=_HIGH,
                 preferred_element_type=jnp.float32)
    xl_ref[...] = xl
    al_ref[...] = jnp.dot(xl, a_ref[...], precision=_HIGH,
                          preferred_element_type=jnp.float32)


def _layermm(h, W2d, A):
    return pl.pallas_call(
        _layermm_body,
        grid=(NB,),
        in_specs=[pl.BlockSpec((BR, HID), lambda i: (i, 0)),
                  pl.BlockSpec((HID, HID), lambda i: (0, 0)),
                  pl.BlockSpec((HID, 16), lambda i: (0, 0))],
        out_specs=[pl.BlockSpec((BR, HID), lambda i: (i, 0)),
                   pl.BlockSpec((BR, 16), lambda i: (i, 0))],
        out_shape=[jax.ShapeDtypeStruct((N, HID), jnp.float32),
                   jax.ShapeDtypeStruct((N, 16), jnp.float32)],
    )(h, W2d, A)


def _post_body(agg_ref, h_ref, b_ref, g_ref, be_ref, o_ref):
    out = agg_ref[...] + b_ref[...] + h_ref[...]
    mu = jnp.mean(out, axis=-1, keepdims=True)
    var = jnp.mean((out - mu) ** 2, axis=-1, keepdims=True)
    out = (out - mu) / jnp.sqrt(var + 1e-5) * g_ref[...] + be_ref[...]
    o_ref[...] = jnp.where(out > 0, out, jnp.exp(out) - 1.0)


def _post(agg, h, b, g, be):
    return pl.pallas_call(
        _post_body,
        grid=(NB,),
        in_specs=[pl.BlockSpec((BR, HID), lambda i: (i, 0)),
                  pl.BlockSpec((BR, HID), lambda i: (i, 0)),
                  pl.BlockSpec((1, HID), lambda i: (0, 0)),
                  pl.BlockSpec((1, HID), lambda i: (0, 0)),
                  pl.BlockSpec((1, HID), lambda i: (0, 0))],
        out_specs=pl.BlockSpec((BR, HID), lambda i: (i, 0)),
        out_shape=jax.ShapeDtypeStruct((N, HID), jnp.float32),
    )(agg, h, b.reshape(1, HID), g.reshape(1, HID), be.reshape(1, HID))


def _pool_body(batch_ref, h_ref, pooled_ref, cnt_ref):
    i = pl.program_id(0)
    b = batch_ref[0, 0, :]
    grp = lax.broadcasted_iota(jnp.int32, (G, BR), 0)
    onehot = jnp.where(b[None, :] == grp, 1.0, 0.0)
    part = jnp.dot(onehot, h_ref[...], precision=_HIGH,
                   preferred_element_type=jnp.float32)
    cpart = jnp.broadcast_to(jnp.sum(onehot, axis=1, keepdims=True), (G, 128))

    @pl.when(i == 0)
    def _():
        pooled_ref[...] = jnp.zeros_like(pooled_ref)
        cnt_ref[...] = jnp.zeros_like(cnt_ref)

    pooled_ref[...] += part
    cnt_ref[...] += cpart


def _pool(batch3, h):
    return pl.pallas_call(
        _pool_body,
        grid=(NB,),
        in_specs=[pl.BlockSpec((1, 1, BR), lambda i: (i, 0, 0)),
                  pl.BlockSpec((BR, HID), lambda i: (i, 0))],
        out_specs=[pl.BlockSpec((G, HID), lambda i: (0, 0)),
                   pl.BlockSpec((G, 128), lambda i: (0, 0))],
        out_shape=[jax.ShapeDtypeStruct((G, HID), jnp.float32),
                   jax.ShapeDtypeStruct((G, 128), jnp.float32)],
    )(batch3, h)


def _head_body(pooled_ref, cnt_ref, w1_ref, b1_ref, w2_ref, b2_ref, o_ref):
    cnt = jnp.maximum(cnt_ref[...][:, :1], 1.0)
    p = pooled_ref[...] / cnt
    z = jax.nn.relu(jnp.dot(p, w1_ref[...], precision=_HIGH,
                            preferred_element_type=jnp.float32) + b1_ref[...])
    o_ref[...] = jnp.dot(z, w2_ref[...], precision=_HIGH,
                         preferred_element_type=jnp.float32) + b2_ref[...]


def _head(pooled, cnt, Wm1, bm1, Wm2, bm2):
    return pl.pallas_call(
        _head_body,
        in_specs=[pl.BlockSpec((G, HID), lambda: (0, 0)),
                  pl.BlockSpec((G, 128), lambda: (0, 0)),
                  pl.BlockSpec((HID, HID // 2), lambda: (0, 0)),
                  pl.BlockSpec((1, HID // 2), lambda: (0, 0)),
                  pl.BlockSpec((HID // 2, NC), lambda: (0, 0)),
                  pl.BlockSpec((1, NC), lambda: (0, 0))],
        out_specs=pl.BlockSpec((G, NC), lambda: (0, 0)),
        out_shape=jax.ShapeDtypeStruct((G, NC), jnp.float32),
    )(pooled, cnt, Wm1, bm1.reshape(1, -1), Wm2, bm2.reshape(1, -1))


NCORE = 2
NTILE = 16
NWORK = NCORE * NTILE
NPT = 320
NPAD = NWORK * NPT
WSA = 128
WSB = 64
GN = 64
NGRP = NPT // GN
ATTN_ROWS = EL + 16
_SC_CP = None
EPADS = EL + 2 * WSA

def _mesh():
    return plsc.VectorSubcoreMesh(core_axis_name="c", subcore_axis_name="s")


def _sc_params():
    return pltpu.CompilerParams(use_tc_tiling_on_sc=False)


_GDN = lax.GatherDimensionNumbers(offset_dims=(), collapsed_slice_dims=(0,),
                                  start_index_map=(0,))


def _bcast_lane(v16, lane):
    idx = jnp.full((16, 1), lane, jnp.int32)
    return lax.gather(v16, idx, _GDN, (1,),
                      mode=lax.GatherScatterMode.PROMISE_IN_BOUNDS)


def _exrow(alw, aldstown, e, nrel):
    a = alw[e, pl.ds(0, 16)] + aldstown[nrel, :]
    a = jnp.where(a > 0, a, 0.2 * a)
    return jnp.exp(a)


def _s2a_body(al_hbm, ald_hbm, idx3_hbm, rp2d_hbm, den_hbm,
              rp_s, idxw, alw, aldstown, den_own, sem1):
    cid = lax.axis_index("c")
    sid = lax.axis_index("s")
    wid = sid * NCORE + cid
    n0 = wid * NPT

    pltpu.sync_copy(rp2d_hbm.at[wid], rp_s)
    pltpu.sync_copy(ald_hbm.at[pl.ds(n0, NPT)], aldstown)

    @pl.loop(0, NPT)
    def _(i):
        den_own[i, :] = jnp.zeros((16,), jnp.float32)

    e_lo = rp_s[0, pl.ds(0, 16)][0]
    e_hi = rp_s[0, pl.ds(NPT, 16)][0]
    abase = (e_lo // 8) * 8
    nwin = lax.div(e_hi - abase + WSA - 1, WSA)

    def win_body(w, carry):
        base = abase + w * WSA
        lo = jnp.maximum(0, e_lo - base)
        r = jnp.minimum(WSA, e_hi - base)
        pltpu.sync_copy(idx3_hbm.at[pl.ds(0, 4), pl.ds(base, WSA)], idxw)
        pltpu.async_copy(al_hbm.at[idxw.at[0]], alw, sem1).wait()

        def sub_body(sb, carry2):
            dst16 = idxw[2, pl.ds(sb * 16, 16)] - n0
            pos0 = sb * 16
            for j in range(16):
                @pl.when(jnp.logical_and(pos0 + j >= lo, pos0 + j < r))
                def _():
                    nrel = dst16[j]
                    ex = _exrow(alw, aldstown, pos0 + j, nrel)
                    plsc.addupdate(den_own.at[nrel, :], ex)
            return carry2

        lax.fori_loop(0, WSA // 16, sub_body, 0)
        return carry

    lax.fori_loop(0, nwin, win_body, 0)
    pltpu.sync_copy(den_own, den_hbm.at[pl.ds(n0, NPT)])


def _s2a(alsrc16, aldst16, idx3, rp2d):
    f = pl.kernel(
        _s2a_body,
        out_type=jax.ShapeDtypeStruct((NPAD, 16), jnp.float32),
        mesh=_mesh(),
        compiler_params=_sc_params(),
        scratch_types=[pltpu.VMEM((1, 336), jnp.int32),
                       pltpu.VMEM((4, WSA), jnp.int32),
                       pltpu.VMEM((WSA, 16), jnp.float32),
                       pltpu.VMEM((NPT, 16), jnp.float32),
                       pltpu.VMEM((NPT, 16), jnp.float32),
                       pltpu.SemaphoreType.DMA],
    )
    return f(alsrc16, aldst16, idx3, rp2d)


def _s2b_body(xl_hbm, al_hbm, ald_hbm, den_hbm, idx3_hbm,
              rp2d_hbm, agg_hbm, attn_hbm,
              rp_s, idxwA, idxwB, permi, attnw, xlwA, xlwB, alwA, alwB,
              aldstown, rden, outstage,
              semxA, semaA, semxB, semaB):
    cid = lax.axis_index("c")
    sid = lax.axis_index("s")
    wid = sid * NCORE + cid
    n0 = wid * NPT

    pltpu.sync_copy(rp2d_hbm.at[wid], rp_s)
    pltpu.sync_copy(ald_hbm.at[pl.ds(n0, NPT)], aldstown)
    pltpu.sync_copy(den_hbm.at[pl.ds(n0, NPT)], rden)

    @pl.loop(0, NPT)
    def _(i):
        rden[i, :] = 1.0 / (rden[i, :] + 1e-16)

    @pl.loop(0, NGRP)
    def _(g):
        gn0 = n0 + g * GN

        @pl.loop(0, GN)
        def _(i):
            @pl.loop(0, HID // 16)
            def _(c):
                outstage[i, pl.ds(c * 16, 16)] = jnp.zeros((16,), jnp.float32)

        e_lo = rp_s[0, pl.ds(g * GN, 16)][0]
        e_hi = rp_s[0, pl.ds((g + 1) * GN, 16)][0]
        abase = (e_lo // 8) * 8
        nwin = lax.div(e_hi - abase + WSB - 1, WSB)

        def issue(w, idxw, xlw, alw, semx, sema):
            base = abase + w * WSB
            pltpu.sync_copy(idx3_hbm.at[pl.ds(0, 4), pl.ds(base, WSB)], idxw)
            pltpu.async_copy(xl_hbm.at[idxw.at[0]], xlw, semx)
            pltpu.async_copy(al_hbm.at[idxw.at[0]], alw, sema)

        def wait(idxw, xlw, alw, semx, sema):
            pltpu.make_async_copy(xl_hbm.at[idxw.at[0]], xlw, semx).wait()
            pltpu.make_async_copy(al_hbm.at[idxw.at[0]], alw, sema).wait()

        def compute(w, idxw, xlw, alw):
            base = abase + w * WSB
            lo = jnp.maximum(0, e_lo - base)
            r = jnp.minimum(WSB, e_hi - base)

            def sub_body(sb, carry2):
                dst16 = idxw[2, pl.ds(sb * 16, 16)] - n0
                pos0 = sb * 16
                for j in range(16):
                    @pl.when(jnp.logical_and(pos0 + j >= lo, pos0 + j < r))
                    def _():
                        nrel = dst16[j]
                        nrel64 = nrel - g * GN
                        e = pos0 + j
                        ex = _exrow(alw, aldstown, e, nrel)
                        a_row = ex * rden[nrel, :]
                        attnw[e, :] = a_row
                        for hh in range(H):
                            bh = _bcast_lane(a_row, hh)
                            for q in range(4):
                                col = hh * C + q * 16
                                plsc.addupdate(
                                    outstage.at[nrel64, pl.ds(col, 16)],
                                    bh * xlw[e, pl.ds(col, 16)])
                return carry2

            lax.fori_loop(0, WSB // 16, sub_body, 0)

            @pl.loop(0, WSB // 16)
            def _(v):
                pos = lax.iota(jnp.int32, 16) + v * 16
                idx = idxw[1, pl.ds(v * 16, 16)]
                keep = jnp.logical_and(pos >= lo, pos < r)
                permi[pl.ds(v * 16, 16)] = jnp.where(keep, idx, EL)
            pltpu.sync_copy(attnw, attn_hbm.at[permi])

        @pl.when(nwin > 0)
        def _():
            issue(0, idxwA, xlwA, alwA, semxA, semaA)

        def pair_body(k, carry):
            w0 = 2 * k
            w1 = w0 + 1
            wait(idxwA, xlwA, alwA, semxA, semaA)

            @pl.when(w1 < nwin)
            def _():
                issue(w1, idxwB, xlwB, alwB, semxB, semaB)

            compute(w0, idxwA, xlwA, alwA)

            @pl.when(w1 < nwin)
            def _():
                wait(idxwB, xlwB, alwB, semxB, semaB)

                @pl.when(w1 + 1 < nwin)
                def _():
                    issue(w1 + 1, idxwA, xlwA, alwA, semxA, semaA)

                compute(w1, idxwB, xlwB, alwB)
            return carry

        lax.fori_loop(0, lax.div(nwin + 1, 2), pair_body, 0)
        pltpu.sync_copy(outstage, agg_hbm.at[pl.ds(gn0, GN)])


def _s2b(xl, alsrc16, aldst16, den, idx3, rp2d):
    f = pl.kernel(
        _s2b_body,
        out_type=[jax.ShapeDtypeStruct((NPAD, HID), jnp.float32),
                  jax.ShapeDtypeStruct((ATTN_ROWS, 16), jnp.float32)],
        mesh=_mesh(),
        compiler_params=_sc_params(),
        scratch_types=[pltpu.VMEM((1, 336), jnp.int32),
                       pltpu.VMEM((4, WSB), jnp.int32),
                       pltpu.VMEM((4, WSB), jnp.int32),
                       pltpu.VMEM((WSB,), jnp.int32),
                       pltpu.VMEM((WSB, 16), jnp.float32),
                       pltpu.VMEM((WSB, HID), jnp.float32),
                       pltpu.VMEM((WSB, HID), jnp.float32),
                       pltpu.VMEM((WSB, 16), jnp.float32),
                       pltpu.VMEM((WSB, 16), jnp.float32),
                       pltpu.VMEM((NPT, 16), jnp.float32),
                       pltpu.VMEM((NPT, 16), jnp.float32),
                       pltpu.VMEM((GN, HID), jnp.float32),
                       pltpu.SemaphoreType.DMA,
                       pltpu.SemaphoreType.DMA,
                       pltpu.SemaphoreType.DMA,
                       pltpu.SemaphoreType.DMA],
    )
    return f(xl, alsrc16, aldst16, den, idx3, rp2d)


def kernel(x, edge_index, batch, Wp, bp,
           W0, as0, ad0, b0, g0, be0,
           W1, as1, ad1, b1, g1, be1,
           W2, as2, ad2, b2, g2, be2,
           Wm1, bm1, Wm2, bm2):
    loops = jnp.arange(N, dtype=edge_index.dtype)
    src = jnp.concatenate([edge_index[0], loops])
    dst = jnp.concatenate([edge_index[1], loops])

    iota = jnp.arange(EL, dtype=jnp.int32)
    dst_s, src_s, perm = lax.sort((dst, src, iota), num_keys=1)
    srcs_p = jnp.concatenate([src_s, jnp.zeros((EPADS - EL,), jnp.int32)])
    perms_p = jnp.concatenate([perm, jnp.full((EPADS - EL,), EL, jnp.int32)])
    dsts_p = jnp.concatenate([dst_s, jnp.full((EPADS - EL,), N, jnp.int32)])
    idx3 = jnp.stack([srcs_p, perms_p, dsts_p, dsts_p])
    rp = jnp.searchsorted(dst_s, jnp.arange(NPAD + 1, dtype=jnp.int32)
                          ).astype(jnp.int32)
    rp_flat = jnp.concatenate([rp, jnp.full((352,), EL, jnp.int32)])
    rp2d = rp_flat[jnp.arange(NWORK)[:, None] * NPT
                   + jnp.arange(336)[None, :]].reshape(NWORK, 1, 336)

    eye = jnp.eye(H, dtype=jnp.float32)
    def mkA(a_s, a_d):
        As = (eye[:, None, :] * a_s[:, :, None]).reshape(HID, H)
        Ad = (eye[:, None, :] * a_d[:, :, None]).reshape(HID, H)
        return jnp.concatenate([As, Ad], axis=1)

    h = _proj(x, Wp, bp)
    attns = []
    for (Wl, a_s, a_d, b, g, be) in ((W0, as0, ad0, b0, g0, be0),
                                     (W1, as1, ad1, b1, g1, be1),
                                     (W2, as2, ad2, b2, g2, be2)):
        xl, al = _layermm(h, Wl.reshape(HID, HID), mkA(a_s, a_d))
        alsrc16 = jnp.pad(al[:, :8], ((0, NPAD - N), (0, 8)))
        aldst16 = jnp.pad(al[:, 8:16], ((0, NPAD - N), (0, 8)))
        den = _s2a(alsrc16, aldst16, idx3, rp2d)
        agg_pad, attn_pad = _s2b(xl, alsrc16, aldst16, den, idx3, rp2d)
        h = _post(agg_pad[:N], h, b, g, be)
        attns.append(attn_pad[:EL, :8])

    pooled, cnt = _pool(batch.reshape(NB, 1, BR), h)
    logits = _head(pooled, cnt, Wm1, bm1, Wm2, bm2)
    return (logits, attns[0], attns[1], attns[2])


# speedup vs baseline: 1.8207x; 1.8207x over previous
import functools

import jax
import jax.numpy as jnp
from jax import lax
from jax.experimental import pallas as pl
from jax.experimental.pallas import tpu as pltpu
from jax.experimental.pallas import tpu_sc as plsc

N = 10000
E = 160000
EL = E + N
DIN = 128
HID = 512
H = 8
C = 64
G = 16
NC = 5
NB = 25
BR = N // NB

_HIGH = lax.Precision.HIGHEST


def _proj_body(x_ref, wp_ref, bp_ref, o_ref):
    o_ref[...] = jax.nn.relu(
        jnp.dot(x_ref[...], wp_ref[...], precision=_HIGH,
                preferred_element_type=jnp.float32) + bp_ref[...])


def _proj(x, Wp, bp):
    return pl.pallas_call(
        _proj_body,
        grid=(NB,),
        in_specs=[pl.BlockSpec((BR, DIN), lambda i: (i, 0)),
                  pl.BlockSpec((DIN, HID), lambda i: (0, 0)),
                  pl.BlockSpec((1, HID), lambda i: (0, 0))],
        out_specs=pl.BlockSpec((BR, HID), lambda i: (i, 0)),
        out_shape=jax.ShapeDtypeStruct((N, HID), jnp.float32),
    )(x, Wp, bp.reshape(1, HID))


def _layermm_body(h_ref, w_ref, a_ref, xl_ref, al_ref):
    xl = jnp.dot(h_ref[...], w_ref[...], precision=_HIGH,
                 preferred_element_type=jnp.float32)
    xl_ref[...] = xl
    al_ref[...] = jnp.dot(xl, a_ref[...], precision=_HIGH,
                          preferred_element_type=jnp.float32)


def _layermm(h, W2d, A):
    return pl.pallas_call(
        _layermm_body,
        grid=(NB,),
        in_specs=[pl.BlockSpec((BR, HID), lambda i: (i, 0)),
                  pl.BlockSpec((HID, HID), lambda i: (0, 0)),
                  pl.BlockSpec((HID, 16), lambda i: (0, 0))],
        out_specs=[pl.BlockSpec((BR, HID), lambda i: (i, 0)),
                   pl.BlockSpec((BR, 16), lambda i: (i, 0))],
        out_shape=[jax.ShapeDtypeStruct((N, HID), jnp.float32),
                   jax.ShapeDtypeStruct((N, 16), jnp.float32)],
    )(h, W2d, A)


def _post_body(agg_ref, h_ref, b_ref, g_ref, be_ref, o_ref):
    out = agg_ref[...] + b_ref[...] + h_ref[...]
    mu = jnp.mean(out, axis=-1, keepdims=True)
    var = jnp.mean((out - mu) ** 2, axis=-1, keepdims=True)
    out = (out - mu) / jnp.sqrt(var + 1e-5) * g_ref[...] + be_ref[...]
    o_ref[...] = jnp.where(out > 0, out, jnp.exp(out) - 1.0)


def _post(agg, h, b, g, be):
    return pl.pallas_call(
        _post_body,
        grid=(NB,),
        in_specs=[pl.BlockSpec((BR, HID), lambda i: (i, 0)),
                  pl.BlockSpec((BR, HID), lambda i: (i, 0)),
                  pl.BlockSpec((1, HID), lambda i: (0, 0)),
                  pl.BlockSpec((1, HID), lambda i: (0, 0)),
                  pl.BlockSpec((1, HID), lambda i: (0, 0))],
        out_specs=pl.BlockSpec((BR, HID), lambda i: (i, 0)),
        out_shape=jax.ShapeDtypeStruct((N, HID), jnp.float32),
    )(agg, h, b.reshape(1, HID), g.reshape(1, HID), be.reshape(1, HID))


def _pool_body(batch_ref, h_ref, pooled_ref, cnt_ref):
    i = pl.program_id(0)
    b = batch_ref[0, 0, :]
    grp = lax.broadcasted_iota(jnp.int32, (G, BR), 0)
    onehot = jnp.where(b[None, :] == grp, 1.0, 0.0)
    part = jnp.dot(onehot, h_ref[...], precision=_HIGH,
                   preferred_element_type=jnp.float32)
    cpart = jnp.broadcast_to(jnp.sum(onehot, axis=1, keepdims=True), (G, 128))

    @pl.when(i == 0)
    def _():
        pooled_ref[...] = jnp.zeros_like(pooled_ref)
        cnt_ref[...] = jnp.zeros_like(cnt_ref)

    pooled_ref[...] += part
    cnt_ref[...] += cpart


def _pool(batch3, h):
    return pl.pallas_call(
        _pool_body,
        grid=(NB,),
        in_specs=[pl.BlockSpec((1, 1, BR), lambda i: (i, 0, 0)),
                  pl.BlockSpec((BR, HID), lambda i: (i, 0))],
        out_specs=[pl.BlockSpec((G, HID), lambda i: (0, 0)),
                   pl.BlockSpec((G, 128), lambda i: (0, 0))],
        out_shape=[jax.ShapeDtypeStruct((G, HID), jnp.float32),
                   jax.ShapeDtypeStruct((G, 128), jnp.float32)],
    )(batch3, h)


def _head_body(pooled_ref, cnt_ref, w1_ref, b1_ref, w2_ref, b2_ref, o_ref):
    cnt = jnp.maximum(cnt_ref[...][:, :1], 1.0)
    p = pooled_ref[...] / cnt
    z = jax.nn.relu(jnp.dot(p, w1_ref[...], precision=_HIGH,
                            preferred_element_type=jnp.float32) + b1_ref[...])
    o_ref[...] = jnp.dot(z, w2_ref[...], precision=_HIGH,
                         preferred_element_type=jnp.float32) + b2_ref[...]


def _head(pooled, cnt, Wm1, bm1, Wm2, bm2):
    return pl.pallas_call(
        _head_body,
        in_specs=[pl.BlockSpec((G, HID), lambda: (0, 0)),
                  pl.BlockSpec((G, 128), lambda: (0, 0)),
                  pl.BlockSpec((HID, HID // 2), lambda: (0, 0)),
                  pl.BlockSpec((1, HID // 2), lambda: (0, 0)),
                  pl.BlockSpec((HID // 2, NC), lambda: (0, 0)),
                  pl.BlockSpec((1, NC), lambda: (0, 0))],
        out_specs=pl.BlockSpec((G, NC), lambda: (0, 0)),
        out_shape=jax.ShapeDtypeStruct((G, NC), jnp.float32),
    )(pooled, cnt, Wm1, bm1.reshape(1, -1), Wm2, bm2.reshape(1, -1))


NCORE = 2
NTILE = 16
NWORK = NCORE * NTILE
NPT = 320
NPAD = NWORK * NPT
WSA = 128
WSB = 64
GN = 64
NGRP = NPT // GN
ATTN_ROWS = EL + 16
_SC_CP = None
EPADS = EL + 2 * WSA

def _mesh():
    return plsc.VectorSubcoreMesh(core_axis_name="c", subcore_axis_name="s")


def _sc_params():
    return pltpu.CompilerParams(use_tc_tiling_on_sc=False)


_GDN = lax.GatherDimensionNumbers(offset_dims=(), collapsed_slice_dims=(0,),
                                  start_index_map=(0,))


def _bcast_lane(v16, lane):
    idx = jnp.full((16, 1), lane, jnp.int32)
    return lax.gather(v16, idx, _GDN, (1,),
                      mode=lax.GatherScatterMode.PROMISE_IN_BOUNDS)


def _exrow(alw, aldstown, e, nrel):
    a = alw[e, pl.ds(0, 16)] + aldstown[nrel, :]
    a = jnp.where(a > 0, a, 0.2 * a)
    return jnp.exp(a)


def _s2a_body(al_hbm, ald_hbm, idx3_hbm, rp2d_hbm, den_hbm,
              rp_s, idxw, alw, aldstown, den_own, sem1):
    cid = lax.axis_index("c")
    sid = lax.axis_index("s")
    wid = sid * NCORE + cid
    n0 = wid * NPT

    pltpu.sync_copy(rp2d_hbm.at[wid], rp_s)
    pltpu.sync_copy(ald_hbm.at[pl.ds(n0, NPT)], aldstown)

    @pl.loop(0, NPT)
    def _(i):
        den_own[i, :] = jnp.zeros((16,), jnp.float32)

    e_lo = rp_s[0, pl.ds(0, 16)][0]
    e_hi = rp_s[0, pl.ds(NPT, 16)][0]
    abase = (e_lo // 8) * 8
    nwin = lax.div(e_hi - abase + WSA - 1, WSA)

    def win_body(w, carry):
        base = abase + w * WSA
        lo = jnp.maximum(0, e_lo - base)
        r = jnp.minimum(WSA, e_hi - base)
        pltpu.sync_copy(idx3_hbm.at[pl.ds(0, 4), pl.ds(base, WSA)], idxw)
        pltpu.async_copy(al_hbm.at[idxw.at[0]], alw, sem1).wait()

        def sub_body(sb, carry2):
            dst16 = idxw[2, pl.ds(sb * 16, 16)] - n0
            pos0 = sb * 16
            for j in range(16):
                @pl.when(jnp.logical_and(pos0 + j >= lo, pos0 + j < r))
                def _():
                    nrel = dst16[j]
                    ex = _exrow(alw, aldstown, pos0 + j, nrel)
                    plsc.addupdate(den_own.at[nrel, :], ex)
            return carry2

        lax.fori_loop(0, WSA // 16, sub_body, 0)
        return carry

    lax.fori_loop(0, nwin, win_body, 0)
    pltpu.sync_copy(den_own, den_hbm.at[pl.ds(n0, NPT)])


def _s2a(alsrc16, aldst16, idx3, rp2d):
    f = pl.kernel(
        _s2a_body,
        out_type=jax.ShapeDtypeStruct((NPAD, 16), jnp.float32),
        mesh=_mesh(),
        compiler_params=_sc_params(),
        scratch_types=[pltpu.VMEM((1, 336), jnp.int32),
                       pltpu.VMEM((4, WSA), jnp.int32),
                       pltpu.VMEM((WSA, 16), jnp.float32),
                       pltpu.VMEM((NPT, 16), jnp.float32),
                       pltpu.VMEM((NPT, 16), jnp.float32),
                       pltpu.SemaphoreType.DMA],
    )
    return f(alsrc16, aldst16, idx3, rp2d)


def _s2b_body(xl_hbm, al_hbm, ald_hbm, den_hbm, idx3_hbm,
              rp2d_hbm, agg_hbm, attn_hbm,
              rp_s, idxw, permi, attnw, xlw, alw, aldstown, rden,
              outstage, sem1, sem2):
    cid = lax.axis_index("c")
    sid = lax.axis_index("s")
    wid = sid * NCORE + cid
    n0 = wid * NPT

    pltpu.sync_copy(rp2d_hbm.at[wid], rp_s)
    pltpu.sync_copy(ald_hbm.at[pl.ds(n0, NPT)], aldstown)
    pltpu.sync_copy(den_hbm.at[pl.ds(n0, NPT)], rden)

    @pl.loop(0, NPT)
    def _(i):
        rden[i, :] = 1.0 / (rden[i, :] + 1e-16)

    @pl.loop(0, NGRP)
    def _(g):
        gn0 = n0 + g * GN

        @pl.loop(0, GN)
        def _(i):
            @pl.loop(0, HID // 16)
            def _(c):
                outstage[i, pl.ds(c * 16, 16)] = jnp.zeros((16,), jnp.float32)

        e_lo = rp_s[0, pl.ds(g * GN, 16)][0]
        e_hi = rp_s[0, pl.ds((g + 1) * GN, 16)][0]
        abase = (e_lo // 8) * 8
        nwin = lax.div(e_hi - abase + WSB - 1, WSB)

        def win_body(w, carry):
            base = abase + w * WSB
            lo = jnp.maximum(0, e_lo - base)
            r = jnp.minimum(WSB, e_hi - base)
            pltpu.sync_copy(idx3_hbm.at[pl.ds(0, 4), pl.ds(base, WSB)],
                            idxw)
            cp1 = pltpu.async_copy(xl_hbm.at[idxw.at[0]], xlw, sem1)
            cp2 = pltpu.async_copy(al_hbm.at[idxw.at[0]], alw, sem2)
            cp1.wait()
            cp2.wait()

            def sub_body(sb, carry2):
                dst16 = idxw[2, pl.ds(sb * 16, 16)] - n0
                pos0 = sb * 16
                for j in range(16):
                    @pl.when(jnp.logical_and(pos0 + j >= lo, pos0 + j < r))
                    def _():
                        nrel = dst16[j]
                        nrel64 = nrel - g * GN
                        e = pos0 + j
                        ex = _exrow(alw, aldstown, e, nrel)
                        a_row = ex * rden[nrel, :]
                        attnw[e, :] = a_row
                        bhs = [_bcast_lane(a_row, hh) for hh in range(H)]
                        vals = [(hh * C + q * 16,
                                 bhs[hh] * xlw[e, pl.ds(hh * C + q * 16, 16)])
                                for hh in range(H) for q in range(4)]
                        for col, v in vals:
                            plsc.addupdate(
                                outstage.at[nrel64, pl.ds(col, 16)], v)
                return carry2

            lax.fori_loop(0, WSB // 16, sub_body, 0)

            @pl.loop(0, WSB // 16)
            def _(v):
                pos = lax.iota(jnp.int32, 16) + v * 16
                idx = idxw[1, pl.ds(v * 16, 16)]
                keep = jnp.logical_and(pos >= lo, pos < r)
                permi[pl.ds(v * 16, 16)] = jnp.where(keep, idx, EL)
            pltpu.sync_copy(attnw, attn_hbm.at[permi])
            return carry

        lax.fori_loop(0, nwin, win_body, 0)
        pltpu.sync_copy(outstage, agg_hbm.at[pl.ds(gn0, GN)])


def _s2b(xl, alsrc16, aldst16, den, idx3, rp2d):
    f = pl.kernel(
        _s2b_body,
        out_type=[jax.ShapeDtypeStruct((NPAD, HID), jnp.float32),
                  jax.ShapeDtypeStruct((ATTN_ROWS, 16), jnp.float32)],
        mesh=_mesh(),
        compiler_params=_sc_params(),
        scratch_types=[pltpu.VMEM((1, 336), jnp.int32),
                       pltpu.VMEM((4, WSB), jnp.int32),
                       pltpu.VMEM((WSB,), jnp.int32),
                       pltpu.VMEM((WSB, 16), jnp.float32),
                       pltpu.VMEM((WSB, HID), jnp.float32),
                       pltpu.VMEM((WSB, 16), jnp.float32),
                       pltpu.VMEM((NPT, 16), jnp.float32),
                       pltpu.VMEM((NPT, 16), jnp.float32),
                       pltpu.VMEM((GN, HID), jnp.float32),
                       pltpu.SemaphoreType.DMA,
                       pltpu.SemaphoreType.DMA],
    )
    return f(xl, alsrc16, aldst16, den, idx3, rp2d)


def kernel(x, edge_index, batch, Wp, bp,
           W0, as0, ad0, b0, g0, be0,
           W1, as1, ad1, b1, g1, be1,
           W2, as2, ad2, b2, g2, be2,
           Wm1, bm1, Wm2, bm2):
    loops = jnp.arange(N, dtype=edge_index.dtype)
    src = jnp.concatenate([edge_index[0], loops])
    dst = jnp.concatenate([edge_index[1], loops])

    iota = jnp.arange(EL, dtype=jnp.int32)
    dst_s, src_s, perm = lax.sort((dst, src, iota), num_keys=1)
    srcs_p = jnp.concatenate([src_s, jnp.zeros((EPADS - EL,), jnp.int32)])
    perms_p = jnp.concatenate([perm, jnp.full((EPADS - EL,), EL, jnp.int32)])
    dsts_p = jnp.concatenate([dst_s, jnp.full((EPADS - EL,), N, jnp.int32)])
    idx3 = jnp.stack([srcs_p, perms_p, dsts_p, dsts_p])
    rp = jnp.searchsorted(dst_s, jnp.arange(NPAD + 1, dtype=jnp.int32)
                          ).astype(jnp.int32)
    rp_flat = jnp.concatenate([rp, jnp.full((352,), EL, jnp.int32)])
    rp2d = rp_flat[jnp.arange(NWORK)[:, None] * NPT
                   + jnp.arange(336)[None, :]].reshape(NWORK, 1, 336)

    eye = jnp.eye(H, dtype=jnp.float32)
    def mkA(a_s, a_d):
        As = (eye[:, None, :] * a_s[:, :, None]).reshape(HID, H)
        Ad = (eye[:, None, :] * a_d[:, :, None]).reshape(HID, H)
        return jnp.concatenate([As, Ad], axis=1)

    h = _proj(x, Wp, bp)
    attns = []
    for (Wl, a_s, a_d, b, g, be) in ((W0, as0, ad0, b0, g0, be0),
                                     (W1, as1, ad1, b1, g1, be1),
                                     (W2, as2, ad2, b2, g2, be2)):
        xl, al = _layermm(h, Wl.reshape(HID, HID), mkA(a_s, a_d))
        alsrc16 = jnp.pad(al[:, :8], ((0, NPAD - N), (0, 8)))
        aldst16 = jnp.pad(al[:, 8:16], ((0, NPAD - N), (0, 8)))
        den = _s2a(alsrc16, aldst16, idx3, rp2d)
        agg_pad, attn_pad = _s2b(xl, alsrc16, aldst16, den, idx3, rp2d)
        h = _post(agg_pad[:N], h, b, g, be)
        attns.append(attn_pad[:EL, :8])

    pooled, cnt = _pool(batch.reshape(NB, 1, BR), h)
    logits = _head(pooled, cnt, Wm1, bm1, Wm2, bm2)
    return (logits, attns[0], attns[1], attns[2])


# speedup vs baseline: 1.9931x; 1.0947x over previous
import functools

import jax
import jax.numpy as jnp
from jax import lax
from jax.experimental import pallas as pl
from jax.experimental.pallas import tpu as pltpu
from jax.experimental.pallas import tpu_sc as plsc

N = 10000
E = 160000
EL = E + N
DIN = 128
HID = 512
H = 8
C = 64
G = 16
NC = 5
NB = 25
BR = N // NB

_HIGH = lax.Precision.HIGHEST


def _proj_body(x_ref, wp_ref, bp_ref, o_ref):
    o_ref[...] = jax.nn.relu(
        jnp.dot(x_ref[...], wp_ref[...], precision=_HIGH,
                preferred_element_type=jnp.float32) + bp_ref[...])


def _proj(x, Wp, bp):
    return pl.pallas_call(
        _proj_body,
        grid=(NB,),
        in_specs=[pl.BlockSpec((BR, DIN), lambda i: (i, 0)),
                  pl.BlockSpec((DIN, HID), lambda i: (0, 0)),
                  pl.BlockSpec((1, HID), lambda i: (0, 0))],
        out_specs=pl.BlockSpec((BR, HID), lambda i: (i, 0)),
        out_shape=jax.ShapeDtypeStruct((N, HID), jnp.float32),
    )(x, Wp, bp.reshape(1, HID))


def _layermm_body(h_ref, w_ref, a_ref, xl_ref, al_ref):
    xl = jnp.dot(h_ref[...], w_ref[...], precision=_HIGH,
                 preferred_element_type=jnp.float32)
    xl_ref[...] = xl
    al_ref[...] = jnp.dot(xl, a_ref[...], precision=_HIGH,
                          preferred_element_type=jnp.float32)


def _layermm(h, W2d, A):
    return pl.pallas_call(
        _layermm_body,
        grid=(NB,),
        in_specs=[pl.BlockSpec((BR, HID), lambda i: (i, 0)),
                  pl.BlockSpec((HID, HID), lambda i: (0, 0)),
                  pl.BlockSpec((HID, 16), lambda i: (0, 0))],
        out_specs=[pl.BlockSpec((BR, HID), lambda i: (i, 0)),
                   pl.BlockSpec((BR, 16), lambda i: (i, 0))],
        out_shape=[jax.ShapeDtypeStruct((N, HID), jnp.float32),
                   jax.ShapeDtypeStruct((N, 16), jnp.float32)],
    )(h, W2d, A)


def _post_body(agg_ref, h_ref, b_ref, g_ref, be_ref, o_ref):
    out = agg_ref[...] + b_ref[...] + h_ref[...]
    mu = jnp.mean(out, axis=-1, keepdims=True)
    var = jnp.mean((out - mu) ** 2, axis=-1, keepdims=True)
    out = (out - mu) / jnp.sqrt(var + 1e-5) * g_ref[...] + be_ref[...]
    o_ref[...] = jnp.where(out > 0, out, jnp.exp(out) - 1.0)


def _post(agg, h, b, g, be):
    return pl.pallas_call(
        _post_body,
        grid=(NB,),
        in_specs=[pl.BlockSpec((BR, HID), lambda i: (i, 0)),
                  pl.BlockSpec((BR, HID), lambda i: (i, 0)),
                  pl.BlockSpec((1, HID), lambda i: (0, 0)),
                  pl.BlockSpec((1, HID), lambda i: (0, 0)),
                  pl.BlockSpec((1, HID), lambda i: (0, 0))],
        out_specs=pl.BlockSpec((BR, HID), lambda i: (i, 0)),
        out_shape=jax.ShapeDtypeStruct((N, HID), jnp.float32),
    )(agg, h, b.reshape(1, HID), g.reshape(1, HID), be.reshape(1, HID))


def _pool_body(batch_ref, h_ref, pooled_ref, cnt_ref):
    i = pl.program_id(0)
    b = batch_ref[0, 0, :]
    grp = lax.broadcasted_iota(jnp.int32, (G, BR), 0)
    onehot = jnp.where(b[None, :] == grp, 1.0, 0.0)
    part = jnp.dot(onehot, h_ref[...], precision=_HIGH,
                   preferred_element_type=jnp.float32)
    cpart = jnp.broadcast_to(jnp.sum(onehot, axis=1, keepdims=True), (G, 128))

    @pl.when(i == 0)
    def _():
        pooled_ref[...] = jnp.zeros_like(pooled_ref)
        cnt_ref[...] = jnp.zeros_like(cnt_ref)

    pooled_ref[...] += part
    cnt_ref[...] += cpart


def _pool(batch3, h):
    return pl.pallas_call(
        _pool_body,
        grid=(NB,),
        in_specs=[pl.BlockSpec((1, 1, BR), lambda i: (i, 0, 0)),
                  pl.BlockSpec((BR, HID), lambda i: (i, 0))],
        out_specs=[pl.BlockSpec((G, HID), lambda i: (0, 0)),
                   pl.BlockSpec((G, 128), lambda i: (0, 0))],
        out_shape=[jax.ShapeDtypeStruct((G, HID), jnp.float32),
                   jax.ShapeDtypeStruct((G, 128), jnp.float32)],
    )(batch3, h)


def _head_body(pooled_ref, cnt_ref, w1_ref, b1_ref, w2_ref, b2_ref, o_ref):
    cnt = jnp.maximum(cnt_ref[...][:, :1], 1.0)
    p = pooled_ref[...] / cnt
    z = jax.nn.relu(jnp.dot(p, w1_ref[...], precision=_HIGH,
                            preferred_element_type=jnp.float32) + b1_ref[...])
    o_ref[...] = jnp.dot(z, w2_ref[...], precision=_HIGH,
                         preferred_element_type=jnp.float32) + b2_ref[...]


def _head(pooled, cnt, Wm1, bm1, Wm2, bm2):
    return pl.pallas_call(
        _head_body,
        in_specs=[pl.BlockSpec((G, HID), lambda: (0, 0)),
                  pl.BlockSpec((G, 128), lambda: (0, 0)),
                  pl.BlockSpec((HID, HID // 2), lambda: (0, 0)),
                  pl.BlockSpec((1, HID // 2), lambda: (0, 0)),
                  pl.BlockSpec((HID // 2, NC), lambda: (0, 0)),
                  pl.BlockSpec((1, NC), lambda: (0, 0))],
        out_specs=pl.BlockSpec((G, NC), lambda: (0, 0)),
        out_shape=jax.ShapeDtypeStruct((G, NC), jnp.float32),
    )(pooled, cnt, Wm1, bm1.reshape(1, -1), Wm2, bm2.reshape(1, -1))


NCORE = 2
NTILE = 16
NWORK = NCORE * NTILE
NPT = 320
NPAD = NWORK * NPT
WSA = 128
WSB = 64
GN = 64
NGRP = NPT // GN
ATTN_ROWS = EL + 16
_SC_CP = None
EPADS = EL + 2 * WSA

def _mesh():
    return plsc.VectorSubcoreMesh(core_axis_name="c", subcore_axis_name="s")


def _sc_params():
    return pltpu.CompilerParams(use_tc_tiling_on_sc=False)


_GDN = lax.GatherDimensionNumbers(offset_dims=(), collapsed_slice_dims=(0,),
                                  start_index_map=(0,))


def _bcast_lane(v16, lane):
    idx = jnp.full((16, 1), lane, jnp.int32)
    return lax.gather(v16, idx, _GDN, (1,),
                      mode=lax.GatherScatterMode.PROMISE_IN_BOUNDS)


def _exrow(alw, aldstown, e, nrel):
    a = alw[e, pl.ds(0, 16)] + aldstown[nrel, :]
    a = jnp.where(a > 0, a, 0.2 * a)
    return jnp.exp(a)


def _s2a_body(al_hbm, ald_hbm, idx3_hbm, rp2d_hbm, den_hbm,
              rp_s, idxw, alw, aldstown, den_own, sem1):
    cid = lax.axis_index("c")
    sid = lax.axis_index("s")
    wid = sid * NCORE + cid
    n0 = wid * NPT

    pltpu.sync_copy(rp2d_hbm.at[wid], rp_s)
    pltpu.sync_copy(ald_hbm.at[pl.ds(n0, NPT)], aldstown)

    @pl.loop(0, NPT)
    def _(i):
        den_own[i, :] = jnp.zeros((16,), jnp.float32)

    e_lo = rp_s[0, pl.ds(0, 16)][0]
    e_hi = rp_s[0, pl.ds(NPT, 16)][0]
    abase = (e_lo // 8) * 8
    nwin = lax.div(e_hi - abase + WSA - 1, WSA)

    def win_body(w, carry):
        base = abase + w * WSA
        lo = jnp.maximum(0, e_lo - base)
        r = jnp.minimum(WSA, e_hi - base)
        pltpu.sync_copy(idx3_hbm.at[pl.ds(0, 4), pl.ds(base, WSA)], idxw)
        pltpu.async_copy(al_hbm.at[idxw.at[0]], alw, sem1).wait()

        def sub_body(sb, carry2):
            dst16 = idxw[2, pl.ds(sb * 16, 16)] - n0
            dst16 = jnp.clip(dst16, 0, NPT - 1)
            pos0 = sb * 16
            exs = []
            for j in range(16):
                nrel = dst16[j]
                valid = jnp.logical_and(pos0 + j >= lo, pos0 + j < r)
                mf = jnp.where(valid, 1.0, 0.0)
                exs.append((nrel,
                            mf * _exrow(alw, aldstown, pos0 + j, nrel)))
            for nrel, ex in exs:
                plsc.addupdate(den_own.at[nrel, :], ex)
            return carry2

        lax.fori_loop(0, WSA // 16, sub_body, 0)
        return carry

    lax.fori_loop(0, nwin, win_body, 0)
    pltpu.sync_copy(den_own, den_hbm.at[pl.ds(n0, NPT)])


def _s2a(alsrc16, aldst16, idx3, rp2d):
    f = pl.kernel(
        _s2a_body,
        out_type=jax.ShapeDtypeStruct((NPAD, 16), jnp.float32),
        mesh=_mesh(),
        compiler_params=_sc_params(),
        scratch_types=[pltpu.VMEM((1, 336), jnp.int32),
                       pltpu.VMEM((4, WSA), jnp.int32),
                       pltpu.VMEM((WSA, 16), jnp.float32),
                       pltpu.VMEM((NPT, 16), jnp.float32),
                       pltpu.VMEM((NPT, 16), jnp.float32),
                       pltpu.SemaphoreType.DMA],
    )
    return f(alsrc16, aldst16, idx3, rp2d)


def _s2b_body(xl_hbm, al_hbm, ald_hbm, den_hbm, idx3_hbm,
              rp2d_hbm, agg_hbm, attn_hbm,
              rp_s, idxw, permi, attnw, xlw, alw, aldstown, rden,
              outstage, sem1, sem2):
    cid = lax.axis_index("c")
    sid = lax.axis_index("s")
    wid = sid * NCORE + cid
    n0 = wid * NPT

    pltpu.sync_copy(rp2d_hbm.at[wid], rp_s)
    pltpu.sync_copy(ald_hbm.at[pl.ds(n0, NPT)], aldstown)
    pltpu.sync_copy(den_hbm.at[pl.ds(n0, NPT)], rden)

    @pl.loop(0, NPT)
    def _(i):
        rden[i, :] = 1.0 / (rden[i, :] + 1e-16)

    @pl.loop(0, NGRP)
    def _(g):
        gn0 = n0 + g * GN

        @pl.loop(0, GN)
        def _(i):
            @pl.loop(0, HID // 16)
            def _(c):
                outstage[i, pl.ds(c * 16, 16)] = jnp.zeros((16,), jnp.float32)

        e_lo = rp_s[0, pl.ds(g * GN, 16)][0]
        e_hi = rp_s[0, pl.ds((g + 1) * GN, 16)][0]
        abase = (e_lo // 8) * 8
        nwin = lax.div(e_hi - abase + WSB - 1, WSB)

        def win_body(w, carry):
            base = abase + w * WSB
            lo = jnp.maximum(0, e_lo - base)
            r = jnp.minimum(WSB, e_hi - base)
            pltpu.sync_copy(idx3_hbm.at[pl.ds(0, 4), pl.ds(base, WSB)],
                            idxw)
            cp1 = pltpu.async_copy(xl_hbm.at[idxw.at[0]], xlw, sem1)
            cp2 = pltpu.async_copy(al_hbm.at[idxw.at[0]], alw, sem2)
            cp1.wait()
            cp2.wait()

            def sub_body(sb, carry2):
                dst16 = idxw[2, pl.ds(sb * 16, 16)] - n0
                dst16 = jnp.clip(dst16, g * GN, g * GN + GN - 1)
                pos0 = sb * 16
                for j in range(16):
                    nrel = dst16[j]
                    nrel64 = nrel - g * GN
                    e = pos0 + j
                    valid = jnp.logical_and(pos0 + j >= lo, pos0 + j < r)
                    mf = jnp.where(valid, 1.0, 0.0)
                    ex = _exrow(alw, aldstown, e, nrel)
                    a_row = ex * rden[nrel, :]
                    attnw[e, :] = a_row
                    a_m = a_row * mf
                    bhs = [_bcast_lane(a_m, hh) for hh in range(H)]
                    vals = [(hh * C + q * 16,
                             bhs[hh] * xlw[e, pl.ds(hh * C + q * 16, 16)])
                            for hh in range(H) for q in range(4)]
                    for col, v in vals:
                        plsc.addupdate(
                            outstage.at[nrel64, pl.ds(col, 16)], v)
                return carry2

            lax.fori_loop(0, WSB // 16, sub_body, 0)

            @pl.loop(0, WSB // 16)
            def _(v):
                pos = lax.iota(jnp.int32, 16) + v * 16
                idx = idxw[1, pl.ds(v * 16, 16)]
                keep = jnp.logical_and(pos >= lo, pos < r)
                permi[pl.ds(v * 16, 16)] = jnp.where(keep, idx, EL)
            pltpu.sync_copy(attnw, attn_hbm.at[permi])
            return carry

        lax.fori_loop(0, nwin, win_body, 0)
        pltpu.sync_copy(outstage, agg_hbm.at[pl.ds(gn0, GN)])


def _s2b(xl, alsrc16, aldst16, den, idx3, rp2d):
    f = pl.kernel(
        _s2b_body,
        out_type=[jax.ShapeDtypeStruct((NPAD, HID), jnp.float32),
                  jax.ShapeDtypeStruct((ATTN_ROWS, 16), jnp.float32)],
        mesh=_mesh(),
        compiler_params=_sc_params(),
        scratch_types=[pltpu.VMEM((1, 336), jnp.int32),
                       pltpu.VMEM((4, WSB), jnp.int32),
                       pltpu.VMEM((WSB,), jnp.int32),
                       pltpu.VMEM((WSB, 16), jnp.float32),
                       pltpu.VMEM((WSB, HID), jnp.float32),
                       pltpu.VMEM((WSB, 16), jnp.float32),
                       pltpu.VMEM((NPT, 16), jnp.float32),
                       pltpu.VMEM((NPT, 16), jnp.float32),
                       pltpu.VMEM((GN, HID), jnp.float32),
                       pltpu.SemaphoreType.DMA,
                       pltpu.SemaphoreType.DMA],
    )
    return f(xl, alsrc16, aldst16, den, idx3, rp2d)


def kernel(x, edge_index, batch, Wp, bp,
           W0, as0, ad0, b0, g0, be0,
           W1, as1, ad1, b1, g1, be1,
           W2, as2, ad2, b2, g2, be2,
           Wm1, bm1, Wm2, bm2):
    loops = jnp.arange(N, dtype=edge_index.dtype)
    src = jnp.concatenate([edge_index[0], loops])
    dst = jnp.concatenate([edge_index[1], loops])

    iota = jnp.arange(EL, dtype=jnp.int32)
    dst_s, src_s, perm = lax.sort((dst, src, iota), num_keys=1)
    srcs_p = jnp.concatenate([src_s, jnp.zeros((EPADS - EL,), jnp.int32)])
    perms_p = jnp.concatenate([perm, jnp.full((EPADS - EL,), EL, jnp.int32)])
    dsts_p = jnp.concatenate([dst_s, jnp.full((EPADS - EL,), N, jnp.int32)])
    idx3 = jnp.stack([srcs_p, perms_p, dsts_p, dsts_p])
    rp = jnp.searchsorted(dst_s, jnp.arange(NPAD + 1, dtype=jnp.int32)
                          ).astype(jnp.int32)
    rp_flat = jnp.concatenate([rp, jnp.full((352,), EL, jnp.int32)])
    rp2d = rp_flat[jnp.arange(NWORK)[:, None] * NPT
                   + jnp.arange(336)[None, :]].reshape(NWORK, 1, 336)

    eye = jnp.eye(H, dtype=jnp.float32)
    def mkA(a_s, a_d):
        As = (eye[:, None, :] * a_s[:, :, None]).reshape(HID, H)
        Ad = (eye[:, None, :] * a_d[:, :, None]).reshape(HID, H)
        return jnp.concatenate([As, Ad], axis=1)

    h = _proj(x, Wp, bp)
    attns = []
    for (Wl, a_s, a_d, b, g, be) in ((W0, as0, ad0, b0, g0, be0),
                                     (W1, as1, ad1, b1, g1, be1),
                                     (W2, as2, ad2, b2, g2, be2)):
        xl, al = _layermm(h, Wl.reshape(HID, HID), mkA(a_s, a_d))
        alsrc16 = jnp.pad(al[:, :8], ((0, NPAD - N), (0, 8)))
        aldst16 = jnp.pad(al[:, 8:16], ((0, NPAD - N), (0, 8)))
        den = _s2a(alsrc16, aldst16, idx3, rp2d)
        agg_pad, attn_pad = _s2b(xl, alsrc16, aldst16, den, idx3, rp2d)
        h = _post(agg_pad[:N], h, b, g, be)
        attns.append(attn_pad[:EL, :8])

    pooled, cnt = _pool(batch.reshape(NB, 1, BR), h)
    logits = _head(pooled, cnt, Wm1, bm1, Wm2, bm2)
    return (logits, attns[0], attns[1], attns[2])


# speedup vs baseline: 2.0518x; 1.0294x over previous
import functools

import jax
import jax.numpy as jnp
from jax import lax
from jax.experimental import pallas as pl
from jax.experimental.pallas import tpu as pltpu
from jax.experimental.pallas import tpu_sc as plsc

N = 10000
E = 160000
EL = E + N
DIN = 128
HID = 512
H = 8
C = 64
G = 16
NC = 5
NB = 25
BR = N // NB

_HIGH = lax.Precision.HIGHEST


def _proj_body(x_ref, wp_ref, bp_ref, o_ref):
    o_ref[...] = jax.nn.relu(
        jnp.dot(x_ref[...], wp_ref[...], precision=_HIGH,
                preferred_element_type=jnp.float32) + bp_ref[...])


def _proj(x, Wp, bp):
    return pl.pallas_call(
        _proj_body,
        grid=(NB,),
        in_specs=[pl.BlockSpec((BR, DIN), lambda i: (i, 0)),
                  pl.BlockSpec((DIN, HID), lambda i: (0, 0)),
                  pl.BlockSpec((1, HID), lambda i: (0, 0))],
        out_specs=pl.BlockSpec((BR, HID), lambda i: (i, 0)),
        out_shape=jax.ShapeDtypeStruct((N, HID), jnp.float32),
    )(x, Wp, bp.reshape(1, HID))


def _layermm_body(h_ref, w_ref, a_ref, xl_ref, al_ref):
    xl = jnp.dot(h_ref[...], w_ref[...], precision=_HIGH,
                 preferred_element_type=jnp.float32)
    xl_ref[...] = xl
    al_ref[...] = jnp.dot(xl, a_ref[...], precision=_HIGH,
                          preferred_element_type=jnp.float32)


def _layermm(h, W2d, A):
    return pl.pallas_call(
        _layermm_body,
        grid=(NB,),
        in_specs=[pl.BlockSpec((BR, HID), lambda i: (i, 0)),
                  pl.BlockSpec((HID, HID), lambda i: (0, 0)),
                  pl.BlockSpec((HID, 16), lambda i: (0, 0))],
        out_specs=[pl.BlockSpec((BR, HID), lambda i: (i, 0)),
                   pl.BlockSpec((BR, 16), lambda i: (i, 0))],
        out_shape=[jax.ShapeDtypeStruct((N, HID), jnp.float32),
                   jax.ShapeDtypeStruct((N, 16), jnp.float32)],
    )(h, W2d, A)


def _post_body(agg_ref, h_ref, b_ref, g_ref, be_ref, o_ref):
    out = agg_ref[...] + b_ref[...] + h_ref[...]
    mu = jnp.mean(out, axis=-1, keepdims=True)
    var = jnp.mean((out - mu) ** 2, axis=-1, keepdims=True)
    out = (out - mu) / jnp.sqrt(var + 1e-5) * g_ref[...] + be_ref[...]
    o_ref[...] = jnp.where(out > 0, out, jnp.exp(out) - 1.0)


def _post(agg, h, b, g, be):
    return pl.pallas_call(
        _post_body,
        grid=(NB,),
        in_specs=[pl.BlockSpec((BR, HID), lambda i: (i, 0)),
                  pl.BlockSpec((BR, HID), lambda i: (i, 0)),
                  pl.BlockSpec((1, HID), lambda i: (0, 0)),
                  pl.BlockSpec((1, HID), lambda i: (0, 0)),
                  pl.BlockSpec((1, HID), lambda i: (0, 0))],
        out_specs=pl.BlockSpec((BR, HID), lambda i: (i, 0)),
        out_shape=jax.ShapeDtypeStruct((N, HID), jnp.float32),
    )(agg, h, b.reshape(1, HID), g.reshape(1, HID), be.reshape(1, HID))


def _pool_body(batch_ref, h_ref, pooled_ref, cnt_ref):
    i = pl.program_id(0)
    b = batch_ref[0, 0, :]
    grp = lax.broadcasted_iota(jnp.int32, (G, BR), 0)
    onehot = jnp.where(b[None, :] == grp, 1.0, 0.0)
    part = jnp.dot(onehot, h_ref[...], precision=_HIGH,
                   preferred_element_type=jnp.float32)
    cpart = jnp.broadcast_to(jnp.sum(onehot, axis=1, keepdims=True), (G, 128))

    @pl.when(i == 0)
    def _():
        pooled_ref[...] = jnp.zeros_like(pooled_ref)
        cnt_ref[...] = jnp.zeros_like(cnt_ref)

    pooled_ref[...] += part
    cnt_ref[...] += cpart


def _pool(batch3, h):
    return pl.pallas_call(
        _pool_body,
        grid=(NB,),
        in_specs=[pl.BlockSpec((1, 1, BR), lambda i: (i, 0, 0)),
                  pl.BlockSpec((BR, HID), lambda i: (i, 0))],
        out_specs=[pl.BlockSpec((G, HID), lambda i: (0, 0)),
                   pl.BlockSpec((G, 128), lambda i: (0, 0))],
        out_shape=[jax.ShapeDtypeStruct((G, HID), jnp.float32),
                   jax.ShapeDtypeStruct((G, 128), jnp.float32)],
    )(batch3, h)


def _head_body(pooled_ref, cnt_ref, w1_ref, b1_ref, w2_ref, b2_ref, o_ref):
    cnt = jnp.maximum(cnt_ref[...][:, :1], 1.0)
    p = pooled_ref[...] / cnt
    z = jax.nn.relu(jnp.dot(p, w1_ref[...], precision=_HIGH,
                            preferred_element_type=jnp.float32) + b1_ref[...])
    o_ref[...] = jnp.dot(z, w2_ref[...], precision=_HIGH,
                         preferred_element_type=jnp.float32) + b2_ref[...]


def _head(pooled, cnt, Wm1, bm1, Wm2, bm2):
    return pl.pallas_call(
        _head_body,
        in_specs=[pl.BlockSpec((G, HID), lambda: (0, 0)),
                  pl.BlockSpec((G, 128), lambda: (0, 0)),
                  pl.BlockSpec((HID, HID // 2), lambda: (0, 0)),
                  pl.BlockSpec((1, HID // 2), lambda: (0, 0)),
                  pl.BlockSpec((HID // 2, NC), lambda: (0, 0)),
                  pl.BlockSpec((1, NC), lambda: (0, 0))],
        out_specs=pl.BlockSpec((G, NC), lambda: (0, 0)),
        out_shape=jax.ShapeDtypeStruct((G, NC), jnp.float32),
    )(pooled, cnt, Wm1, bm1.reshape(1, -1), Wm2, bm2.reshape(1, -1))


NCORE = 2
NTILE = 16
NWORK = NCORE * NTILE
NPT = 320
NPAD = NWORK * NPT
WSA = 128
WSB = 128
GN = 64
NGRP = NPT // GN
ATTN_ROWS = EL + 16
_SC_CP = None
EPADS = EL + 2 * WSA

def _mesh():
    return plsc.VectorSubcoreMesh(core_axis_name="c", subcore_axis_name="s")


def _sc_params():
    return pltpu.CompilerParams(use_tc_tiling_on_sc=False)


_GDN = lax.GatherDimensionNumbers(offset_dims=(), collapsed_slice_dims=(0,),
                                  start_index_map=(0,))


def _bcast_lane(v16, lane):
    idx = jnp.full((16, 1), lane, jnp.int32)
    return lax.gather(v16, idx, _GDN, (1,),
                      mode=lax.GatherScatterMode.PROMISE_IN_BOUNDS)


def _exrow(alw, aldstown, e, nrel):
    a = alw[e, pl.ds(0, 16)] + aldstown[nrel, :]
    a = jnp.where(a > 0, a, 0.2 * a)
    return jnp.exp(a)


def _s2a_body(al_hbm, ald_hbm, idx3_hbm, rp2d_hbm, den_hbm,
              rp_s, idxw, alw, aldstown, den_own, sem1):
    cid = lax.axis_index("c")
    sid = lax.axis_index("s")
    wid = sid * NCORE + cid
    n0 = wid * NPT

    pltpu.sync_copy(rp2d_hbm.at[wid], rp_s)
    pltpu.sync_copy(ald_hbm.at[pl.ds(n0, NPT)], aldstown)

    @pl.loop(0, NPT)
    def _(i):
        den_own[i, :] = jnp.zeros((16,), jnp.float32)

    e_lo = rp_s[0, pl.ds(0, 16)][0]
    e_hi = rp_s[0, pl.ds(NPT, 16)][0]
    abase = (e_lo // 8) * 8
    nwin = lax.div(e_hi - abase + WSA - 1, WSA)

    def win_body(w, carry):
        base = abase + w * WSA
        lo = jnp.maximum(0, e_lo - base)
        r = jnp.minimum(WSA, e_hi - base)
        pltpu.sync_copy(idx3_hbm.at[pl.ds(0, 4), pl.ds(base, WSA)], idxw)
        pltpu.async_copy(al_hbm.at[idxw.at[0]], alw, sem1).wait()

        def sub_body(sb, carry2):
            dst16 = idxw[2, pl.ds(sb * 16, 16)] - n0
            dst16 = jnp.clip(dst16, 0, NPT - 1)
            pos0 = sb * 16
            exs = []
            for j in range(16):
                nrel = dst16[j]
                valid = jnp.logical_and(pos0 + j >= lo, pos0 + j < r)
                mf = jnp.where(valid, 1.0, 0.0)
                exs.append((nrel,
                            mf * _exrow(alw, aldstown, pos0 + j, nrel)))
            for nrel, ex in exs:
                plsc.addupdate(den_own.at[nrel, :], ex)
            return carry2

        lax.fori_loop(0, WSA // 16, sub_body, 0)
        return carry

    lax.fori_loop(0, nwin, win_body, 0)
    pltpu.sync_copy(den_own, den_hbm.at[pl.ds(n0, NPT)])


def _s2a(alsrc16, aldst16, idx3, rp2d):
    f = pl.kernel(
        _s2a_body,
        out_type=jax.ShapeDtypeStruct((NPAD, 16), jnp.float32),
        mesh=_mesh(),
        compiler_params=_sc_params(),
        scratch_types=[pltpu.VMEM((1, 336), jnp.int32),
                       pltpu.VMEM((4, WSA), jnp.int32),
                       pltpu.VMEM((WSA, 16), jnp.float32),
                       pltpu.VMEM((NPT, 16), jnp.float32),
                       pltpu.VMEM((NPT, 16), jnp.float32),
                       pltpu.SemaphoreType.DMA],
    )
    return f(alsrc16, aldst16, idx3, rp2d)


def _s2b_body(xl_hbm, al_hbm, ald_hbm, den_hbm, idx3_hbm,
              rp2d_hbm, agg_hbm, attn_hbm,
              rp_s, idxw, permi, attnw, xlw, alw, aldstown, rden,
              outstage, sem1, sem2):
    cid = lax.axis_index("c")
    sid = lax.axis_index("s")
    wid = sid * NCORE + cid
    n0 = wid * NPT

    pltpu.sync_copy(rp2d_hbm.at[wid], rp_s)
    pltpu.sync_copy(ald_hbm.at[pl.ds(n0, NPT)], aldstown)
    pltpu.sync_copy(den_hbm.at[pl.ds(n0, NPT)], rden)

    @pl.loop(0, NPT)
    def _(i):
        rden[i, :] = 1.0 / (rden[i, :] + 1e-16)

    @pl.loop(0, NGRP)
    def _(g):
        gn0 = n0 + g * GN

        @pl.loop(0, GN)
        def _(i):
            @pl.loop(0, HID // 16)
            def _(c):
                outstage[i, pl.ds(c * 16, 16)] = jnp.zeros((16,), jnp.float32)

        e_lo = rp_s[0, pl.ds(g * GN, 16)][0]
        e_hi = rp_s[0, pl.ds((g + 1) * GN, 16)][0]
        abase = (e_lo // 8) * 8
        nwin = lax.div(e_hi - abase + WSB - 1, WSB)

        def win_body(w, carry):
            base = abase + w * WSB
            lo = jnp.maximum(0, e_lo - base)
            r = jnp.minimum(WSB, e_hi - base)
            pltpu.sync_copy(idx3_hbm.at[pl.ds(0, 4), pl.ds(base, WSB)],
                            idxw)
            cp1 = pltpu.async_copy(xl_hbm.at[idxw.at[0]], xlw, sem1)
            cp2 = pltpu.async_copy(al_hbm.at[idxw.at[0]], alw, sem2)
            cp1.wait()
            cp2.wait()

            def sub_body(sb, carry2):
                dst16 = idxw[2, pl.ds(sb * 16, 16)] - n0
                dst16 = jnp.clip(dst16, g * GN, g * GN + GN - 1)
                pos0 = sb * 16
                for j in range(16):
                    nrel = dst16[j]
                    nrel64 = nrel - g * GN
                    e = pos0 + j
                    valid = jnp.logical_and(pos0 + j >= lo, pos0 + j < r)
                    mf = jnp.where(valid, 1.0, 0.0)
                    ex = _exrow(alw, aldstown, e, nrel)
                    a_row = ex * rden[nrel, :]
                    attnw[e, :] = a_row
                    a_m = a_row * mf
                    bhs = [_bcast_lane(a_m, hh) for hh in range(H)]
                    vals = [(hh * C + q * 16,
                             bhs[hh] * xlw[e, pl.ds(hh * C + q * 16, 16)])
                            for hh in range(H) for q in range(4)]
                    for col, v in vals:
                        plsc.addupdate(
                            outstage.at[nrel64, pl.ds(col, 16)], v)
                return carry2

            lax.fori_loop(0, WSB // 16, sub_body, 0)

            @pl.loop(0, WSB // 16)
            def _(v):
                pos = lax.iota(jnp.int32, 16) + v * 16
                idx = idxw[1, pl.ds(v * 16, 16)]
                keep = jnp.logical_and(pos >= lo, pos < r)
                permi[pl.ds(v * 16, 16)] = jnp.where(keep, idx, EL)
            pltpu.sync_copy(attnw, attn_hbm.at[permi])
            return carry

        lax.fori_loop(0, nwin, win_body, 0)
        pltpu.sync_copy(outstage, agg_hbm.at[pl.ds(gn0, GN)])


def _s2b(xl, alsrc16, aldst16, den, idx3, rp2d):
    f = pl.kernel(
        _s2b_body,
        out_type=[jax.ShapeDtypeStruct((NPAD, HID), jnp.float32),
                  jax.ShapeDtypeStruct((ATTN_ROWS, 16), jnp.float32)],
        mesh=_mesh(),
        compiler_params=_sc_params(),
        scratch_types=[pltpu.VMEM((1, 336), jnp.int32),
                       pltpu.VMEM((4, WSB), jnp.int32),
                       pltpu.VMEM((WSB,), jnp.int32),
                       pltpu.VMEM((WSB, 16), jnp.float32),
                       pltpu.VMEM((WSB, HID), jnp.float32),
                       pltpu.VMEM((WSB, 16), jnp.float32),
                       pltpu.VMEM((NPT, 16), jnp.float32),
                       pltpu.VMEM((NPT, 16), jnp.float32),
                       pltpu.VMEM((GN, HID), jnp.float32),
                       pltpu.SemaphoreType.DMA,
                       pltpu.SemaphoreType.DMA],
    )
    return f(xl, alsrc16, aldst16, den, idx3, rp2d)


def kernel(x, edge_index, batch, Wp, bp,
           W0, as0, ad0, b0, g0, be0,
           W1, as1, ad1, b1, g1, be1,
           W2, as2, ad2, b2, g2, be2,
           Wm1, bm1, Wm2, bm2):
    loops = jnp.arange(N, dtype=edge_index.dtype)
    src = jnp.concatenate([edge_index[0], loops])
    dst = jnp.concatenate([edge_index[1], loops])

    eid = jnp.arange(EL, dtype=jnp.uint32)
    key = (dst.astype(jnp.uint32) << 18) | eid
    key_s, src_s = lax.sort((key, src), num_keys=1)
    dst_s = (key_s >> 18).astype(jnp.int32)
    perm = (key_s & jnp.uint32((1 << 18) - 1)).astype(jnp.int32)
    srcs_p = jnp.concatenate([src_s, jnp.zeros((EPADS - EL,), jnp.int32)])
    perms_p = jnp.concatenate([perm, jnp.full((EPADS - EL,), EL, jnp.int32)])
    dsts_p = jnp.concatenate([dst_s, jnp.full((EPADS - EL,), N, jnp.int32)])
    idx3 = jnp.stack([srcs_p, perms_p, dsts_p, dsts_p])
    rp = jnp.searchsorted(dst_s, jnp.arange(NPAD + 1, dtype=jnp.int32)
                          ).astype(jnp.int32)
    rp_flat = jnp.concatenate([rp, jnp.full((352,), EL, jnp.int32)])
    rp2d = rp_flat[jnp.arange(NWORK)[:, None] * NPT
                   + jnp.arange(336)[None, :]].reshape(NWORK, 1, 336)

    eye = jnp.eye(H, dtype=jnp.float32)
    def mkA(a_s, a_d):
        As = (eye[:, None, :] * a_s[:, :, None]).reshape(HID, H)
        Ad = (eye[:, None, :] * a_d[:, :, None]).reshape(HID, H)
        return jnp.concatenate([As, Ad], axis=1)

    h = _proj(x, Wp, bp)
    attns = []
    for (Wl, a_s, a_d, b, g, be) in ((W0, as0, ad0, b0, g0, be0),
                                     (W1, as1, ad1, b1, g1, be1),
                                     (W2, as2, ad2, b2, g2, be2)):
        xl, al = _layermm(h, Wl.reshape(HID, HID), mkA(a_s, a_d))
        alsrc16 = jnp.pad(al[:, :8], ((0, NPAD - N), (0, 8)))
        aldst16 = jnp.pad(al[:, 8:16], ((0, NPAD - N), (0, 8)))
        den = _s2a(alsrc16, aldst16, idx3, rp2d)
        agg_pad, attn_pad = _s2b(xl, alsrc16, aldst16, den, idx3, rp2d)
        h = _post(agg_pad[:N], h, b, g, be)
        attns.append(attn_pad[:EL, :8])

    pooled, cnt = _pool(batch.reshape(NB, 1, BR), h)
    logits = _head(pooled, cnt, Wm1, bm1, Wm2, bm2)
    return (logits, attns[0], attns[1], attns[2])


# speedup vs baseline: 2.1555x; 1.0505x over previous
import functools

import jax
import jax.numpy as jnp
from jax import lax
from jax.experimental import pallas as pl
from jax.experimental.pallas import tpu as pltpu
from jax.experimental.pallas import tpu_sc as plsc

N = 10000
E = 160000
EL = E + N
DIN = 128
HID = 512
H = 8
C = 64
G = 16
NC = 5
NB = 25
BR = N // NB

_HIGH = lax.Precision.HIGHEST


def _proj_body(x_ref, wp_ref, bp_ref, o_ref):
    o_ref[...] = jax.nn.relu(
        jnp.dot(x_ref[...], wp_ref[...], precision=_HIGH,
                preferred_element_type=jnp.float32) + bp_ref[...])


def _proj(x, Wp, bp):
    return pl.pallas_call(
        _proj_body,
        grid=(NB,),
        in_specs=[pl.BlockSpec((BR, DIN), lambda i: (i, 0)),
                  pl.BlockSpec((DIN, HID), lambda i: (0, 0)),
                  pl.BlockSpec((1, HID), lambda i: (0, 0))],
        out_specs=pl.BlockSpec((BR, HID), lambda i: (i, 0)),
        out_shape=jax.ShapeDtypeStruct((N, HID), jnp.float32),
    )(x, Wp, bp.reshape(1, HID))


def _layermm_body(h_ref, w_ref, a_ref, xl_ref, al_ref):
    xl = jnp.dot(h_ref[...], w_ref[...], precision=_HIGH,
                 preferred_element_type=jnp.float32)
    xl_ref[...] = xl
    al_ref[...] = jnp.dot(xl, a_ref[...], precision=_HIGH,
                          preferred_element_type=jnp.float32)


def _layermm(h, W2d, A):
    return pl.pallas_call(
        _layermm_body,
        grid=(NB,),
        in_specs=[pl.BlockSpec((BR, HID), lambda i: (i, 0)),
                  pl.BlockSpec((HID, HID), lambda i: (0, 0)),
                  pl.BlockSpec((HID, 16), lambda i: (0, 0))],
        out_specs=[pl.BlockSpec((BR, HID), lambda i: (i, 0)),
                   pl.BlockSpec((BR, 16), lambda i: (i, 0))],
        out_shape=[jax.ShapeDtypeStruct((N, HID), jnp.float32),
                   jax.ShapeDtypeStruct((N, 16), jnp.float32)],
    )(h, W2d, A)


def _post_body(agg_ref, h_ref, b_ref, g_ref, be_ref, o_ref):
    out = agg_ref[...] + b_ref[...] + h_ref[...]
    mu = jnp.mean(out, axis=-1, keepdims=True)
    var = jnp.mean((out - mu) ** 2, axis=-1, keepdims=True)
    out = (out - mu) / jnp.sqrt(var + 1e-5) * g_ref[...] + be_ref[...]
    o_ref[...] = jnp.where(out > 0, out, jnp.exp(out) - 1.0)


def _post(agg, h, b, g, be):
    return pl.pallas_call(
        _post_body,
        grid=(NB,),
        in_specs=[pl.BlockSpec((BR, HID), lambda i: (i, 0)),
                  pl.BlockSpec((BR, HID), lambda i: (i, 0)),
                  pl.BlockSpec((1, HID), lambda i: (0, 0)),
                  pl.BlockSpec((1, HID), lambda i: (0, 0)),
                  pl.BlockSpec((1, HID), lambda i: (0, 0))],
        out_specs=pl.BlockSpec((BR, HID), lambda i: (i, 0)),
        out_shape=jax.ShapeDtypeStruct((N, HID), jnp.float32),
    )(agg, h, b.reshape(1, HID), g.reshape(1, HID), be.reshape(1, HID))


def _pool_body(batch_ref, h_ref, pooled_ref, cnt_ref):
    i = pl.program_id(0)
    b = batch_ref[0, 0, :]
    grp = lax.broadcasted_iota(jnp.int32, (G, BR), 0)
    onehot = jnp.where(b[None, :] == grp, 1.0, 0.0)
    part = jnp.dot(onehot, h_ref[...], precision=_HIGH,
                   preferred_element_type=jnp.float32)
    cpart = jnp.broadcast_to(jnp.sum(onehot, axis=1, keepdims=True), (G, 128))

    @pl.when(i == 0)
    def _():
        pooled_ref[...] = jnp.zeros_like(pooled_ref)
        cnt_ref[...] = jnp.zeros_like(cnt_ref)

    pooled_ref[...] += part
    cnt_ref[...] += cpart


def _pool(batch3, h):
    return pl.pallas_call(
        _pool_body,
        grid=(NB,),
        in_specs=[pl.BlockSpec((1, 1, BR), lambda i: (i, 0, 0)),
                  pl.BlockSpec((BR, HID), lambda i: (i, 0))],
        out_specs=[pl.BlockSpec((G, HID), lambda i: (0, 0)),
                   pl.BlockSpec((G, 128), lambda i: (0, 0))],
        out_shape=[jax.ShapeDtypeStruct((G, HID), jnp.float32),
                   jax.ShapeDtypeStruct((G, 128), jnp.float32)],
    )(batch3, h)


def _head_body(pooled_ref, cnt_ref, w1_ref, b1_ref, w2_ref, b2_ref, o_ref):
    cnt = jnp.maximum(cnt_ref[...][:, :1], 1.0)
    p = pooled_ref[...] / cnt
    z = jax.nn.relu(jnp.dot(p, w1_ref[...], precision=_HIGH,
                            preferred_element_type=jnp.float32) + b1_ref[...])
    o_ref[...] = jnp.dot(z, w2_ref[...], precision=_HIGH,
                         preferred_element_type=jnp.float32) + b2_ref[...]


def _head(pooled, cnt, Wm1, bm1, Wm2, bm2):
    return pl.pallas_call(
        _head_body,
        in_specs=[pl.BlockSpec((G, HID), lambda: (0, 0)),
                  pl.BlockSpec((G, 128), lambda: (0, 0)),
                  pl.BlockSpec((HID, HID // 2), lambda: (0, 0)),
                  pl.BlockSpec((1, HID // 2), lambda: (0, 0)),
                  pl.BlockSpec((HID // 2, NC), lambda: (0, 0)),
                  pl.BlockSpec((1, NC), lambda: (0, 0))],
        out_specs=pl.BlockSpec((G, NC), lambda: (0, 0)),
        out_shape=jax.ShapeDtypeStruct((G, NC), jnp.float32),
    )(pooled, cnt, Wm1, bm1.reshape(1, -1), Wm2, bm2.reshape(1, -1))


NCORE = 2
NTILE = 16
NWORK = NCORE * NTILE
NPT = 320
NPAD = NWORK * NPT
WSA = 128
WSB = 64
GN = 64
NGRP = NPT // GN
ATTN_ROWS = EL + 16
_SC_CP = None
EPADS = EL + 2 * WSA

def _mesh():
    return plsc.VectorSubcoreMesh(core_axis_name="c", subcore_axis_name="s")


def _sc_params():
    return pltpu.CompilerParams(use_tc_tiling_on_sc=False)


_GDN = lax.GatherDimensionNumbers(offset_dims=(), collapsed_slice_dims=(0,),
                                  start_index_map=(0,))


def _bcast_lane(v16, lane):
    idx = jnp.full((16, 1), lane, jnp.int32)
    return lax.gather(v16, idx, _GDN, (1,),
                      mode=lax.GatherScatterMode.PROMISE_IN_BOUNDS)


def _exrow(alw, aldstown, e, nrel):
    a = alw[e, pl.ds(0, 16)] + aldstown[nrel, :]
    a = jnp.where(a > 0, a, 0.2 * a)
    return jnp.exp(a)


def _s2a_body(al_hbm, ald_hbm, idx3_hbm, rp2d_hbm, den_hbm,
              rp_s, idxw, alw, aldstown, den_own, sem1):
    cid = lax.axis_index("c")
    sid = lax.axis_index("s")
    wid = sid * NCORE + cid
    n0 = wid * NPT

    pltpu.sync_copy(rp2d_hbm.at[wid], rp_s)
    pltpu.sync_copy(ald_hbm.at[pl.ds(n0, NPT)], aldstown)

    @pl.loop(0, NPT)
    def _(i):
        den_own[i, :] = jnp.zeros((16,), jnp.float32)

    e_lo = rp_s[0, pl.ds(0, 16)][0]
    e_hi = rp_s[0, pl.ds(NPT, 16)][0]
    abase = (e_lo // 8) * 8
    nwin = lax.div(e_hi - abase + WSA - 1, WSA)

    def win_body(w, carry):
        base = abase + w * WSA
        lo = jnp.maximum(0, e_lo - base)
        r = jnp.minimum(WSA, e_hi - base)
        pltpu.sync_copy(idx3_hbm.at[pl.ds(0, 4), pl.ds(base, WSA)], idxw)
        pltpu.async_copy(al_hbm.at[idxw.at[0]], alw, sem1).wait()

        def sub_body(sb, carry2):
            dst16 = idxw[2, pl.ds(sb * 16, 16)] - n0
            dst16 = jnp.clip(dst16, 0, NPT - 1)
            pos0 = sb * 16
            exs = []
            for j in range(16):
                nrel = dst16[j]
                valid = jnp.logical_and(pos0 + j >= lo, pos0 + j < r)
                mf = jnp.where(valid, 1.0, 0.0)
                exs.append((nrel,
                            mf * _exrow(alw, aldstown, pos0 + j, nrel)))
            for nrel, ex in exs:
                plsc.addupdate(den_own.at[nrel, :], ex)
            return carry2

        lax.fori_loop(0, WSA // 16, sub_body, 0)
        return carry

    lax.fori_loop(0, nwin, win_body, 0)
    pltpu.sync_copy(den_own, den_hbm.at[pl.ds(n0, NPT)])


def _s2a(alsrc16, aldst16, idx3, rp2d):
    f = pl.kernel(
        _s2a_body,
        out_type=jax.ShapeDtypeStruct((NPAD, 16), jnp.float32),
        mesh=_mesh(),
        compiler_params=_sc_params(),
        scratch_types=[pltpu.VMEM((1, 336), jnp.int32),
                       pltpu.VMEM((4, WSA), jnp.int32),
                       pltpu.VMEM((WSA, 16), jnp.float32),
                       pltpu.VMEM((NPT, 16), jnp.float32),
                       pltpu.VMEM((NPT, 16), jnp.float32),
                       pltpu.SemaphoreType.DMA],
    )
    return f(alsrc16, aldst16, idx3, rp2d)


def _s2b_body(xl_hbm, al_hbm, ald_hbm, den_hbm, idx3_hbm,
              rp2d_hbm, agg_hbm, attn_hbm,
              rp_s, idxwA, idxwB, permi, attnw, xlwA, xlwB, alwA, alwB,
              aldstown, rden, outstage,
              semxA, semaA, semxB, semaB):
    cid = lax.axis_index("c")
    sid = lax.axis_index("s")
    wid = sid * NCORE + cid
    n0 = wid * NPT

    pltpu.sync_copy(rp2d_hbm.at[wid], rp_s)
    pltpu.sync_copy(ald_hbm.at[pl.ds(n0, NPT)], aldstown)
    pltpu.sync_copy(den_hbm.at[pl.ds(n0, NPT)], rden)

    @pl.loop(0, NPT)
    def _(i):
        rden[i, :] = 1.0 / (rden[i, :] + 1e-16)

    @pl.loop(0, NGRP)
    def _(g):
        gn0 = n0 + g * GN

        @pl.loop(0, GN)
        def _(i):
            @pl.loop(0, HID // 16)
            def _(c):
                outstage[i, pl.ds(c * 16, 16)] = jnp.zeros((16,), jnp.float32)

        e_lo = rp_s[0, pl.ds(g * GN, 16)][0]
        e_hi = rp_s[0, pl.ds((g + 1) * GN, 16)][0]
        abase = (e_lo // 8) * 8
        nwin = lax.div(e_hi - abase + WSB - 1, WSB)

        def issue(w, idxw, xlw, alw, semx, sema):
            base = abase + w * WSB
            pltpu.sync_copy(idx3_hbm.at[pl.ds(0, 4), pl.ds(base, WSB)], idxw)
            pltpu.async_copy(xl_hbm.at[idxw.at[0]], xlw, semx)
            pltpu.async_copy(al_hbm.at[idxw.at[0]], alw, sema)

        def wait(idxw, xlw, alw, semx, sema):
            pltpu.make_async_copy(xl_hbm.at[idxw.at[0]], xlw, semx).wait()
            pltpu.make_async_copy(al_hbm.at[idxw.at[0]], alw, sema).wait()

        def compute(w, idxw, xlw, alw):
            base = abase + w * WSB
            lo = jnp.maximum(0, e_lo - base)
            r = jnp.minimum(WSB, e_hi - base)

            def sub_body(sb, carry2):
                dst16 = idxw[2, pl.ds(sb * 16, 16)] - n0
                dst16 = jnp.clip(dst16, g * GN, g * GN + GN - 1)
                pos0 = sb * 16
                for j in range(16):
                    nrel = dst16[j]
                    nrel64 = nrel - g * GN
                    e = pos0 + j
                    valid = jnp.logical_and(pos0 + j >= lo, pos0 + j < r)
                    mf = jnp.where(valid, 1.0, 0.0)
                    ex = _exrow(alw, aldstown, e, nrel)
                    a_row = ex * rden[nrel, :]
                    attnw[e, :] = a_row
                    a_m = a_row * mf
                    bhs = [_bcast_lane(a_m, hh) for hh in range(H)]
                    vals = [(hh * C + q * 16,
                             bhs[hh] * xlw[e, pl.ds(hh * C + q * 16, 16)])
                            for hh in range(H) for q in range(4)]
                    for col, v in vals:
                        plsc.addupdate(
                            outstage.at[nrel64, pl.ds(col, 16)], v)
                return carry2

            lax.fori_loop(0, WSB // 16, sub_body, 0)

            @pl.loop(0, WSB // 16)
            def _(v):
                pos = lax.iota(jnp.int32, 16) + v * 16
                idx = idxw[1, pl.ds(v * 16, 16)]
                keep = jnp.logical_and(pos >= lo, pos < r)
                permi[pl.ds(v * 16, 16)] = jnp.where(keep, idx, EL)
            pltpu.sync_copy(attnw, attn_hbm.at[permi])

        @pl.when(nwin > 0)
        def _():
            issue(0, idxwA, xlwA, alwA, semxA, semaA)

        def pair_body(k, carry):
            w0 = 2 * k
            w1 = w0 + 1
            wait(idxwA, xlwA, alwA, semxA, semaA)

            @pl.when(w1 < nwin)
            def _():
                issue(w1, idxwB, xlwB, alwB, semxB, semaB)

            compute(w0, idxwA, xlwA, alwA)

            @pl.when(w1 < nwin)
            def _():
                wait(idxwB, xlwB, alwB, semxB, semaB)

                @pl.when(w1 + 1 < nwin)
                def _():
                    issue(w1 + 1, idxwA, xlwA, alwA, semxA, semaA)

                compute(w1, idxwB, xlwB, alwB)
            return carry

        lax.fori_loop(0, lax.div(nwin + 1, 2), pair_body, 0)
        pltpu.sync_copy(outstage, agg_hbm.at[pl.ds(gn0, GN)])


def _s2b(xl, alsrc16, aldst16, den, idx3, rp2d):
    f = pl.kernel(
        _s2b_body,
        out_type=[jax.ShapeDtypeStruct((NPAD, HID), jnp.float32),
                  jax.ShapeDtypeStruct((ATTN_ROWS, 16), jnp.float32)],
        mesh=_mesh(),
        compiler_params=_sc_params(),
        scratch_types=[pltpu.VMEM((1, 336), jnp.int32),
                       pltpu.VMEM((4, WSB), jnp.int32),
                       pltpu.VMEM((4, WSB), jnp.int32),
                       pltpu.VMEM((WSB,), jnp.int32),
                       pltpu.VMEM((WSB, 16), jnp.float32),
                       pltpu.VMEM((WSB, HID), jnp.float32),
                       pltpu.VMEM((WSB, HID), jnp.float32),
                       pltpu.VMEM((WSB, 16), jnp.float32),
                       pltpu.VMEM((WSB, 16), jnp.float32),
                       pltpu.VMEM((NPT, 16), jnp.float32),
                       pltpu.VMEM((NPT, 16), jnp.float32),
                       pltpu.VMEM((GN, HID), jnp.float32),
                       pltpu.SemaphoreType.DMA,
                       pltpu.SemaphoreType.DMA,
                       pltpu.SemaphoreType.DMA,
                       pltpu.SemaphoreType.DMA],
    )
    return f(xl, alsrc16, aldst16, den, idx3, rp2d)


def kernel(x, edge_index, batch, Wp, bp,
           W0, as0, ad0, b0, g0, be0,
           W1, as1, ad1, b1, g1, be1,
           W2, as2, ad2, b2, g2, be2,
           Wm1, bm1, Wm2, bm2):
    loops = jnp.arange(N, dtype=edge_index.dtype)
    src = jnp.concatenate([edge_index[0], loops])
    dst = jnp.concatenate([edge_index[1], loops])

    eid = jnp.arange(EL, dtype=jnp.uint32)
    key = (dst.astype(jnp.uint32) << 18) | eid
    key_s, src_s = lax.sort((key, src), num_keys=1)
    dst_s = (key_s >> 18).astype(jnp.int32)
    perm = (key_s & jnp.uint32((1 << 18) - 1)).astype(jnp.int32)
    srcs_p = jnp.concatenate([src_s, jnp.zeros((EPADS - EL,), jnp.int32)])
    perms_p = jnp.concatenate([perm, jnp.full((EPADS - EL,), EL, jnp.int32)])
    dsts_p = jnp.concatenate([dst_s, jnp.full((EPADS - EL,), N, jnp.int32)])
    idx3 = jnp.stack([srcs_p, perms_p, dsts_p, dsts_p])
    rp = jnp.searchsorted(dst_s, jnp.arange(NPAD + 1, dtype=jnp.int32)
                          ).astype(jnp.int32)
    rp_flat = jnp.concatenate([rp, jnp.full((352,), EL, jnp.int32)])
    rp2d = rp_flat[jnp.arange(NWORK)[:, None] * NPT
                   + jnp.arange(336)[None, :]].reshape(NWORK, 1, 336)

    eye = jnp.eye(H, dtype=jnp.float32)
    def mkA(a_s, a_d):
        As = (eye[:, None, :] * a_s[:, :, None]).reshape(HID, H)
        Ad = (eye[:, None, :] * a_d[:, :, None]).reshape(HID, H)
        return jnp.concatenate([As, Ad], axis=1)

    h = _proj(x, Wp, bp)
    attns = []
    for (Wl, a_s, a_d, b, g, be) in ((W0, as0, ad0, b0, g0, be0),
                                     (W1, as1, ad1, b1, g1, be1),
                                     (W2, as2, ad2, b2, g2, be2)):
        xl, al = _layermm(h, Wl.reshape(HID, HID), mkA(a_s, a_d))
        alsrc16 = jnp.pad(al[:, :8], ((0, NPAD - N), (0, 8)))
        aldst16 = jnp.pad(al[:, 8:16], ((0, NPAD - N), (0, 8)))
        den = _s2a(alsrc16, aldst16, idx3, rp2d)
        agg_pad, attn_pad = _s2b(xl, alsrc16, aldst16, den, idx3, rp2d)
        h = _post(agg_pad[:N], h, b, g, be)
        attns.append(attn_pad[:EL, :8])

    pooled, cnt = _pool(batch.reshape(NB, 1, BR), h)
    logits = _head(pooled, cnt, Wm1, bm1, Wm2, bm2)
    return (logits, attns[0], attns[1], attns[2])


# speedup vs baseline: 2.1558x; 1.0001x over previous
import jax
import jax.numpy as jnp
from jax import lax
from jax.experimental import pallas as pl
from jax.experimental.pallas import tpu as pltpu
from jax.experimental.pallas import tpu_sc as plsc

N = 10000
E = 160000
EL = E + N
DIN = 128
HID = 512
H = 8
C = 64
G = 16
NC = 5
NB = 25
BR = N // NB

_HIGH = lax.Precision.HIGHEST


def _proj_body(x_ref, wp_ref, bp_ref, o_ref):
    o_ref[...] = jax.nn.relu(
        jnp.dot(x_ref[...], wp_ref[...], precision=_HIGH,
                preferred_element_type=jnp.float32) + bp_ref[...])


def _proj(x, Wp, bp):
    return pl.pallas_call(
        _proj_body,
        grid=(NB,),
        in_specs=[pl.BlockSpec((BR, DIN), lambda i: (i, 0)),
                  pl.BlockSpec((DIN, HID), lambda i: (0, 0)),
                  pl.BlockSpec((1, HID), lambda i: (0, 0))],
        out_specs=pl.BlockSpec((BR, HID), lambda i: (i, 0)),
        out_shape=jax.ShapeDtypeStruct((N, HID), jnp.float32),
    )(x, Wp, bp.reshape(1, HID))


def _layermm_body(h_ref, w_ref, a_ref, xl_ref, al_ref):
    xl = jnp.dot(h_ref[...], w_ref[...], precision=_HIGH,
                 preferred_element_type=jnp.float32)
    xl_ref[...] = xl
    al_ref[...] = jnp.dot(xl, a_ref[...], precision=_HIGH,
                          preferred_element_type=jnp.float32)


def _layermm(h, W2d, A):
    return pl.pallas_call(
        _layermm_body,
        grid=(NB,),
        in_specs=[pl.BlockSpec((BR, HID), lambda i: (i, 0)),
                  pl.BlockSpec((HID, HID), lambda i: (0, 0)),
                  pl.BlockSpec((HID, 16), lambda i: (0, 0))],
        out_specs=[pl.BlockSpec((BR, HID), lambda i: (i, 0)),
                   pl.BlockSpec((BR, 16), lambda i: (i, 0))],
        out_shape=[jax.ShapeDtypeStruct((N, HID), jnp.float32),
                   jax.ShapeDtypeStruct((N, 16), jnp.float32)],
    )(h, W2d, A)


def _post_body(agg_ref, h_ref, b_ref, g_ref, be_ref, o_ref):
    out = agg_ref[...] + b_ref[...] + h_ref[...]
    mu = jnp.mean(out, axis=-1, keepdims=True)
    var = jnp.mean((out - mu) ** 2, axis=-1, keepdims=True)
    out = (out - mu) / jnp.sqrt(var + 1e-5) * g_ref[...] + be_ref[...]
    o_ref[...] = jnp.where(out > 0, out, jnp.exp(out) - 1.0)


def _post(agg, h, b, g, be):
    return pl.pallas_call(
        _post_body,
        grid=(NB,),
        in_specs=[pl.BlockSpec((BR, HID), lambda i: (i, 0)),
                  pl.BlockSpec((BR, HID), lambda i: (i, 0)),
                  pl.BlockSpec((1, HID), lambda i: (0, 0)),
                  pl.BlockSpec((1, HID), lambda i: (0, 0)),
                  pl.BlockSpec((1, HID), lambda i: (0, 0))],
        out_specs=pl.BlockSpec((BR, HID), lambda i: (i, 0)),
        out_shape=jax.ShapeDtypeStruct((N, HID), jnp.float32),
    )(agg, h, b.reshape(1, HID), g.reshape(1, HID), be.reshape(1, HID))


def _pool_body(batch_ref, h_ref, pooled_ref, cnt_ref):
    i = pl.program_id(0)
    b = batch_ref[0, 0, :]
    grp = lax.broadcasted_iota(jnp.int32, (G, BR), 0)
    onehot = jnp.where(b[None, :] == grp, 1.0, 0.0)
    part = jnp.dot(onehot, h_ref[...], precision=_HIGH,
                   preferred_element_type=jnp.float32)
    cpart = jnp.broadcast_to(jnp.sum(onehot, axis=1, keepdims=True), (G, 128))

    @pl.when(i == 0)
    def _():
        pooled_ref[...] = jnp.zeros_like(pooled_ref)
        cnt_ref[...] = jnp.zeros_like(cnt_ref)

    pooled_ref[...] += part
    cnt_ref[...] += cpart


def _pool(batch3, h):
    return pl.pallas_call(
        _pool_body,
        grid=(NB,),
        in_specs=[pl.BlockSpec((1, 1, BR), lambda i: (i, 0, 0)),
                  pl.BlockSpec((BR, HID), lambda i: (i, 0))],
        out_specs=[pl.BlockSpec((G, HID), lambda i: (0, 0)),
                   pl.BlockSpec((G, 128), lambda i: (0, 0))],
        out_shape=[jax.ShapeDtypeStruct((G, HID), jnp.float32),
                   jax.ShapeDtypeStruct((G, 128), jnp.float32)],
    )(batch3, h)


def _head_body(pooled_ref, cnt_ref, w1_ref, b1_ref, w2_ref, b2_ref, o_ref):
    cnt = jnp.maximum(cnt_ref[...][:, :1], 1.0)
    p = pooled_ref[...] / cnt
    z = jax.nn.relu(jnp.dot(p, w1_ref[...], precision=_HIGH,
                            preferred_element_type=jnp.float32) + b1_ref[...])
    o_ref[...] = jnp.dot(z, w2_ref[...], precision=_HIGH,
                         preferred_element_type=jnp.float32) + b2_ref[...]


def _head(pooled, cnt, Wm1, bm1, Wm2, bm2):
    return pl.pallas_call(
        _head_body,
        in_specs=[pl.BlockSpec((G, HID), lambda: (0, 0)),
                  pl.BlockSpec((G, 128), lambda: (0, 0)),
                  pl.BlockSpec((HID, HID // 2), lambda: (0, 0)),
                  pl.BlockSpec((1, HID // 2), lambda: (0, 0)),
                  pl.BlockSpec((HID // 2, NC), lambda: (0, 0)),
                  pl.BlockSpec((1, NC), lambda: (0, 0))],
        out_specs=pl.BlockSpec((G, NC), lambda: (0, 0)),
        out_shape=jax.ShapeDtypeStruct((G, NC), jnp.float32),
    )(pooled, cnt, Wm1, bm1.reshape(1, -1), Wm2, bm2.reshape(1, -1))


NCORE = 2
NTILE = 16
NWORK = NCORE * NTILE
NPT = 320
NPAD = NWORK * NPT
WSA = 128
WSB = 64
GN = 64
NGRP = NPT // GN
ATTN_ROWS = EL + 16
EPADS = EL + 2 * WSA

def _mesh():
    return plsc.VectorSubcoreMesh(core_axis_name="c", subcore_axis_name="s")


def _sc_params():
    return pltpu.CompilerParams(use_tc_tiling_on_sc=False)


_GDN = lax.GatherDimensionNumbers(offset_dims=(), collapsed_slice_dims=(0,),
                                  start_index_map=(0,))


def _bcast_lane(v16, lane):
    idx = jnp.full((16, 1), lane, jnp.int32)
    return lax.gather(v16, idx, _GDN, (1,),
                      mode=lax.GatherScatterMode.PROMISE_IN_BOUNDS)


def _exrow(alw, aldstown, e, nrel):
    a = alw[e, pl.ds(0, 16)] + aldstown[nrel, :]
    a = jnp.where(a > 0, a, 0.2 * a)
    return jnp.exp(a)


def _s2a_body(al_hbm, ald_hbm, idx3_hbm, rp2d_hbm, den_hbm,
              rp_s, idxw, alw, aldstown, den_own, sem1):
    cid = lax.axis_index("c")
    sid = lax.axis_index("s")
    wid = sid * NCORE + cid
    n0 = wid * NPT

    pltpu.sync_copy(rp2d_hbm.at[wid], rp_s)
    pltpu.sync_copy(ald_hbm.at[pl.ds(n0, NPT)], aldstown)

    @pl.loop(0, NPT)
    def _(i):
        den_own[i, :] = jnp.zeros((16,), jnp.float32)

    e_lo = rp_s[0, pl.ds(0, 16)][0]
    e_hi = rp_s[0, pl.ds(NPT, 16)][0]
    abase = (e_lo // 8) * 8
    nwin = lax.div(e_hi - abase + WSA - 1, WSA)

    def win_body(w, carry):
        base = abase + w * WSA
        lo = jnp.maximum(0, e_lo - base)
        r = jnp.minimum(WSA, e_hi - base)
        pltpu.sync_copy(idx3_hbm.at[pl.ds(0, 4), pl.ds(base, WSA)], idxw)
        pltpu.async_copy(al_hbm.at[idxw.at[0]], alw, sem1).wait()

        def sub_body(sb, carry2):
            dst16 = idxw[2, pl.ds(sb * 16, 16)] - n0
            dst16 = jnp.clip(dst16, 0, NPT - 1)
            pos0 = sb * 16
            exs = []
            for j in range(16):
                nrel = dst16[j]
                valid = jnp.logical_and(pos0 + j >= lo, pos0 + j < r)
                mf = jnp.where(valid, 1.0, 0.0)
                exs.append((nrel,
                            mf * _exrow(alw, aldstown, pos0 + j, nrel)))
            for nrel, ex in exs:
                plsc.addupdate(den_own.at[nrel, :], ex)
            return carry2

        lax.fori_loop(0, WSA // 16, sub_body, 0)
        return carry

    lax.fori_loop(0, nwin, win_body, 0)
    pltpu.sync_copy(den_own, den_hbm.at[pl.ds(n0, NPT)])


def _s2a(alsrc16, aldst16, idx3, rp2d):
    f = pl.kernel(
        _s2a_body,
        out_type=jax.ShapeDtypeStruct((NPAD, 16), jnp.float32),
        mesh=_mesh(),
        compiler_params=_sc_params(),
        scratch_types=[pltpu.VMEM((1, 336), jnp.int32),
                       pltpu.VMEM((4, WSA), jnp.int32),
                       pltpu.VMEM((WSA, 16), jnp.float32),
                       pltpu.VMEM((NPT, 16), jnp.float32),
                       pltpu.VMEM((NPT, 16), jnp.float32),
                       pltpu.SemaphoreType.DMA],
    )
    return f(alsrc16, aldst16, idx3, rp2d)


def _s2b_body(xl_hbm, al_hbm, ald_hbm, den_hbm, idx3_hbm,
              rp2d_hbm, agg_hbm, attn_hbm,
              rp_s, idxwA, idxwB, permi, attnw, xlwA, xlwB, alwA, alwB,
              aldstown, rden, outstage,
              semxA, semaA, semxB, semaB):
    cid = lax.axis_index("c")
    sid = lax.axis_index("s")
    wid = sid * NCORE + cid
    n0 = wid * NPT

    pltpu.sync_copy(rp2d_hbm.at[wid], rp_s)
    pltpu.sync_copy(ald_hbm.at[pl.ds(n0, NPT)], aldstown)
    pltpu.sync_copy(den_hbm.at[pl.ds(n0, NPT)], rden)

    @pl.loop(0, NPT)
    def _(i):
        rden[i, :] = 1.0 / (rden[i, :] + 1e-16)

    @pl.loop(0, NGRP)
    def _(g):
        gn0 = n0 + g * GN

        @pl.loop(0, GN)
        def _(i):
            @pl.loop(0, HID // 16)
            def _(c):
                outstage[i, pl.ds(c * 16, 16)] = jnp.zeros((16,), jnp.float32)

        e_lo = rp_s[0, pl.ds(g * GN, 16)][0]
        e_hi = rp_s[0, pl.ds((g + 1) * GN, 16)][0]
        abase = (e_lo // 8) * 8
        nwin = lax.div(e_hi - abase + WSB - 1, WSB)

        def issue(w, idxw, xlw, alw, semx, sema):
            base = abase + w * WSB
            pltpu.sync_copy(idx3_hbm.at[pl.ds(0, 4), pl.ds(base, WSB)], idxw)
            pltpu.async_copy(xl_hbm.at[idxw.at[0]], xlw, semx)
            pltpu.async_copy(al_hbm.at[idxw.at[0]], alw, sema)

        def wait(idxw, xlw, alw, semx, sema):
            pltpu.make_async_copy(xl_hbm.at[idxw.at[0]], xlw, semx).wait()
            pltpu.make_async_copy(al_hbm.at[idxw.at[0]], alw, sema).wait()

        def compute(w, idxw, xlw, alw):
            base = abase + w * WSB
            lo = jnp.maximum(0, e_lo - base)
            r = jnp.minimum(WSB, e_hi - base)

            def sub_body(sb, carry2):
                dst16 = idxw[2, pl.ds(sb * 16, 16)] - n0
                dst16 = jnp.clip(dst16, g * GN, g * GN + GN - 1)
                pos0 = sb * 16
                for j in range(16):
                    nrel = dst16[j]
                    nrel64 = nrel - g * GN
                    e = pos0 + j
                    valid = jnp.logical_and(pos0 + j >= lo, pos0 + j < r)
                    mf = jnp.where(valid, 1.0, 0.0)
                    ex = _exrow(alw, aldstown, e, nrel)
                    a_row = ex * rden[nrel, :]
                    attnw[e, :] = a_row
                    a_m = a_row * mf
                    bhs = [_bcast_lane(a_m, hh) for hh in range(H)]
                    vals = [(hh * C + q * 16,
                             bhs[hh] * xlw[e, pl.ds(hh * C + q * 16, 16)])
                            for hh in range(H) for q in range(4)]
                    for col, v in vals:
                        plsc.addupdate(
                            outstage.at[nrel64, pl.ds(col, 16)], v)
                return carry2

            lax.fori_loop(0, WSB // 16, sub_body, 0)

            @pl.loop(0, WSB // 16)
            def _(v):
                pos = lax.iota(jnp.int32, 16) + v * 16
                idx = idxw[1, pl.ds(v * 16, 16)]
                keep = jnp.logical_and(pos >= lo, pos < r)
                permi[pl.ds(v * 16, 16)] = jnp.where(keep, idx, EL)
            pltpu.sync_copy(attnw, attn_hbm.at[permi])

        @pl.when(nwin > 0)
        def _():
            issue(0, idxwA, xlwA, alwA, semxA, semaA)

        def pair_body(k, carry):
            w0 = 2 * k
            w1 = w0 + 1
            wait(idxwA, xlwA, alwA, semxA, semaA)

            @pl.when(w1 < nwin)
            def _():
                issue(w1, idxwB, xlwB, alwB, semxB, semaB)

            compute(w0, idxwA, xlwA, alwA)

            @pl.when(w1 < nwin)
            def _():
                wait(idxwB, xlwB, alwB, semxB, semaB)

                @pl.when(w1 + 1 < nwin)
                def _():
                    issue(w1 + 1, idxwA, xlwA, alwA, semxA, semaA)

                compute(w1, idxwB, xlwB, alwB)
            return carry

        lax.fori_loop(0, lax.div(nwin + 1, 2), pair_body, 0)
        pltpu.sync_copy(outstage, agg_hbm.at[pl.ds(gn0, GN)])


def _s2b(xl, alsrc16, aldst16, den, idx3, rp2d):
    f = pl.kernel(
        _s2b_body,
        out_type=[jax.ShapeDtypeStruct((NPAD, HID), jnp.float32),
                  jax.ShapeDtypeStruct((ATTN_ROWS, 16), jnp.float32)],
        mesh=_mesh(),
        compiler_params=_sc_params(),
        scratch_types=[pltpu.VMEM((1, 336), jnp.int32),
                       pltpu.VMEM((4, WSB), jnp.int32),
                       pltpu.VMEM((4, WSB), jnp.int32),
                       pltpu.VMEM((WSB,), jnp.int32),
                       pltpu.VMEM((WSB, 16), jnp.float32),
                       pltpu.VMEM((WSB, HID), jnp.float32),
                       pltpu.VMEM((WSB, HID), jnp.float32),
                       pltpu.VMEM((WSB, 16), jnp.float32),
                       pltpu.VMEM((WSB, 16), jnp.float32),
                       pltpu.VMEM((NPT, 16), jnp.float32),
                       pltpu.VMEM((NPT, 16), jnp.float32),
                       pltpu.VMEM((GN, HID), jnp.float32),
                       pltpu.SemaphoreType.DMA,
                       pltpu.SemaphoreType.DMA,
                       pltpu.SemaphoreType.DMA,
                       pltpu.SemaphoreType.DMA],
    )
    return f(xl, alsrc16, aldst16, den, idx3, rp2d)


def kernel(x, edge_index, batch, Wp, bp,
           W0, as0, ad0, b0, g0, be0,
           W1, as1, ad1, b1, g1, be1,
           W2, as2, ad2, b2, g2, be2,
           Wm1, bm1, Wm2, bm2):
    loops = jnp.arange(N, dtype=edge_index.dtype)
    src = jnp.concatenate([edge_index[0], loops])
    dst = jnp.concatenate([edge_index[1], loops])

    eid = jnp.arange(EL, dtype=jnp.uint32)
    key = (dst.astype(jnp.uint32) << 18) | eid
    key_s, src_s = lax.sort((key, src), num_keys=1)
    dst_s = (key_s >> 18).astype(jnp.int32)
    perm = (key_s & jnp.uint32((1 << 18) - 1)).astype(jnp.int32)
    srcs_p = jnp.concatenate([src_s, jnp.zeros((EPADS - EL,), jnp.int32)])
    perms_p = jnp.concatenate([perm, jnp.full((EPADS - EL,), EL, jnp.int32)])
    dsts_p = jnp.concatenate([dst_s, jnp.full((EPADS - EL,), N, jnp.int32)])
    idx3 = jnp.stack([srcs_p, perms_p, dsts_p, dsts_p])
    rp = jnp.searchsorted(dst_s, jnp.arange(NPAD + 1, dtype=jnp.int32)
                          ).astype(jnp.int32)
    rp_flat = jnp.concatenate([rp, jnp.full((352,), EL, jnp.int32)])
    rp2d = rp_flat[jnp.arange(NWORK)[:, None] * NPT
                   + jnp.arange(336)[None, :]].reshape(NWORK, 1, 336)

    eye = jnp.eye(H, dtype=jnp.float32)
    def mkA(a_s, a_d):
        As = (eye[:, None, :] * a_s[:, :, None]).reshape(HID, H)
        Ad = (eye[:, None, :] * a_d[:, :, None]).reshape(HID, H)
        return jnp.concatenate([As, Ad], axis=1)

    h = _proj(x, Wp, bp)
    attns = []
    for (Wl, a_s, a_d, b, g, be) in ((W0, as0, ad0, b0, g0, be0),
                                     (W1, as1, ad1, b1, g1, be1),
                                     (W2, as2, ad2, b2, g2, be2)):
        xl, al = _layermm(h, Wl.reshape(HID, HID), mkA(a_s, a_d))
        alsrc16 = jnp.pad(al[:, :8], ((0, NPAD - N), (0, 8)))
        aldst16 = jnp.pad(al[:, 8:16], ((0, NPAD - N), (0, 8)))
        den = _s2a(alsrc16, aldst16, idx3, rp2d)
        agg_pad, attn_pad = _s2b(xl, alsrc16, aldst16, den, idx3, rp2d)
        h = _post(agg_pad[:N], h, b, g, be)
        attns.append(attn_pad[:EL, :8])

    pooled, cnt = _pool(batch.reshape(NB, 1, BR), h)
    logits = _head(pooled, cnt, Wm1, bm1, Wm2, bm2)
    return (logits, attns[0], attns[1], attns[2])
